# Initial kernel scaffold; baseline (speedup 1.0000x reference)
import functools

import jax
import jax.numpy as jnp
from jax import lax
from jax.experimental import pallas as pl
from jax.experimental.pallas import tpu as pltpu
from jax.experimental.pallas import tpu_sc as plsc

N = 10000
NP = 10240
E = 320000
D = 128
NC = 2
NS = 16
CH = 80
ECHUNKS = E // (NC * NS) // CH
GCHUNKS = N // CH

_MESH = plsc.VectorSubcoreMesh(
    core_axis_name="c", subcore_axis_name="s", num_cores=NC, num_subcores=NS)


def _zero_vmem_rows(buf, nrows):
    z16 = jnp.zeros((16,), jnp.float32)

    def body(i, _):
        for j in range(D // 16):
            buf[i, pl.ds(j * 16, 16)] = z16
        return 0

    lax.fori_loop(0, nrows, body, 0)


@functools.partial(
    pl.kernel,
    out_type=[
        jax.ShapeDtypeStruct((NC * NP,), jnp.float32),
        jax.ShapeDtypeStruct((N, D), jnp.float32),
        jax.ShapeDtypeStruct((N, D), jnp.float32),
    ],
    mesh=_MESH,
    scratch_types=[
        pltpu.VMEM((CH,), jnp.int32),
        pltpu.VMEM((CH,), jnp.float32),
        pltpu.VMEM((CH, D), jnp.float32),
        pltpu.VMEM_SHARED((NP,), jnp.float32),
    ],
)
def _sc_pre(dst_hbm, w_hbm, x_hbm, p1_hbm, p2_hbm,
            deg_hbm, xp1_hbm, xp2_hbm,
            idxb, valb, rows, dacc):
    cid = lax.axis_index("c")
    sid = lax.axis_index("s")
    wid = sid * NC + cid
    zslice = NP // NS

    valb[...] = jnp.zeros((CH,), jnp.float32)

    def zbody(i, _):
        pltpu.sync_copy(valb, dacc.at[pl.ds(sid * zslice + i * CH, CH)])
        return 0

    lax.fori_loop(0, zslice // CH, zbody, 0)
    plsc.subcore_barrier()

    base_e = (cid * NS + sid) * (ECHUNKS * CH)

    def ebody(c, _):
        off = base_e + c * CH
        pltpu.sync_copy(dst_hbm.at[pl.ds(off, CH)], idxb)
        pltpu.sync_copy(w_hbm.at[pl.ds(off, CH)], valb)
        pltpu.sync_copy(valb, dacc.at[idxb], add=True)
        return 0

    lax.fori_loop(0, ECHUNKS, ebody, 0)
    plsc.subcore_barrier()

    pltpu.sync_copy(dacc.at[pl.ds(sid * zslice, zslice)],
                    deg_hbm.at[pl.ds(cid * NP + sid * zslice, zslice)])

    for i in range((GCHUNKS + NC * NS - 1) // (NC * NS)):
        c = wid + i * NC * NS

        @pl.when(c < GCHUNKS)
        def _():
            off = c * CH
            pltpu.sync_copy(p1_hbm.at[pl.ds(off, CH)], idxb)
            pltpu.sync_copy(x_hbm.at[idxb], rows)
            pltpu.sync_copy(rows, xp1_hbm.at[pl.ds(off, CH)])
            pltpu.sync_copy(p2_hbm.at[pl.ds(off, CH)], idxb)
            pltpu.sync_copy(x_hbm.at[idxb], rows)
            pltpu.sync_copy(rows, xp2_hbm.at[pl.ds(off, CH)])


@functools.partial(
    pl.kernel,
    out_type=[jax.ShapeDtypeStruct((NC * NP, D), jnp.float32)
              for _ in range(4)],
    mesh=_MESH,
    scratch_types=[
        pltpu.VMEM((CH,), jnp.int32),
        pltpu.VMEM((CH,), jnp.int32),
        pltpu.VMEM((CH,), jnp.float32),
        pltpu.VMEM((CH, D), jnp.float32),
        pltpu.VMEM((CH, D), jnp.float32),
        pltpu.VMEM_SHARED((NP, D), jnp.float32),
    ],
)
def _sc_mp(src_hbm, dst_hbm, w_hbm, hs0, hs1, hs2, hs3,
           part0, part1, part2, part3,
           idx_s, idx_d, wv, rows, zrows, acc):
    cid = lax.axis_index("c")
    sid = lax.axis_index("s")
    zslice = NP // NS

    _zero_vmem_rows(zrows, CH)

    hs_list = (hs0, hs1, hs2, hs3)
    part_list = (part0, part1, part2, part3)

    for s in range(4):
        hs = hs_list[s]
        part = part_list[s]

        for j in range(zslice // CH):
            pltpu.sync_copy(zrows, acc.at[pl.ds(sid * zslice + j * CH, CH)])
        plsc.subcore_barrier()

        base_e = (cid * NS + sid) * (ECHUNKS * CH)

        def ebody(c, _):
            off = base_e + c * CH
            pltpu.sync_copy(src_hbm.at[pl.ds(off, CH)], idx_s)
            pltpu.sync_copy(dst_hbm.at[pl.ds(off, CH)], idx_d)
            pltpu.sync_copy(w_hbm.at[pl.ds(off, CH)], wv)
            pltpu.sync_copy(hs.at[idx_s], rows)

            def sbody(k, _2):
                wvec = plsc.load_gather(wv, [jnp.full((16,), k, jnp.int32)])
                for j in range(D // 16):
                    sl = pl.ds(j * 16, 16)
                    rows[k, sl] = rows[k, sl] * wvec
                return 0

            lax.fori_loop(0, CH, sbody, 0)
            pltpu.sync_copy(rows, acc.at[idx_d], add=True)
            return 0

        lax.fori_loop(0, ECHUNKS, ebody, 0)
        plsc.subcore_barrier()

        for j in range(zslice // CH):
            roff = sid * zslice + j * CH
            pltpu.sync_copy(acc.at[pl.ds(roff, CH)],
                            part.at[pl.ds(cid * NP + roff, CH)])
        plsc.subcore_barrier()


def _prelu(z, a):
    return jnp.maximum(z, 0.0) + a * jnp.minimum(z, 0.0)


def _tc_a_body(d0, d1, x, xp1, xp2, w1a, w1b,
               hs0, hs1, hs2, hs3, dinv):
    dv = lax.rsqrt(d0[...] + d1[...] + 1.0)
    dinv[...] = dv
    hs0[...] = jnp.dot(x[...], w1a[...], preferred_element_type=jnp.float32) * dv
    hs1[...] = jnp.dot(x[...], w1b[...], preferred_element_type=jnp.float32) * dv
    hs2[...] = jnp.dot(xp1[...], w1a[...], preferred_element_type=jnp.float32) * dv
    hs3[...] = jnp.dot(xp2[...], w1b[...], preferred_element_type=jnp.float32) * dv


def _tc_a(d0, d1, x, xp1, xp2, w1a, w1b):
    outs = [jax.ShapeDtypeStruct((NP, D), jnp.float32) for _ in range(4)]
    outs.append(jax.ShapeDtypeStruct((NP, 1), jnp.float32))
    return pl.pallas_call(
        _tc_a_body,
        out_shape=outs,
    )(d0, d1, x, xp1, xp2, w1a, w1b)


_RB = 2048
_GRID = NP // _RB


def _tc_b_body(p0a, p0b, p1a, p1b, p2a, p2b, p3a, p3b,
               h0, h1, h2, h3, dinv, bst, ast, wst,
               o0, o1, o2, o3):
    dv = dinv[...]
    hs = (h0, h1, h2, h3)
    pa = (p0a, p1a, p2a, p3a)
    pb = (p0b, p1b, p2b, p3b)
    outs = (o0, o1, o2, o3)
    for s in range(4):
        agg = dv * (pa[s][...] + pb[s][...] + hs[s][...]) + bst[s:s + 1, :]
        z = _prelu(agg, ast[s:s + 1, :])
        outs[s][...] = jnp.dot(z, wst[s],
                               preferred_element_type=jnp.float32) * dv


def _tc_b(parts, hs, dinv, bst, ast, wst):
    bs_part = [pl.BlockSpec((_RB, D), lambda i, o=o: (i + o * _GRID, 0))
               for o in (0, 1)]
    bs_rows = pl.BlockSpec((_RB, D), lambda i: (i, 0))
    in_specs = []
    args = []
    for p in parts:
        args.extend([p, p])
        in_specs.extend(bs_part)
    args.extend(hs)
    in_specs.extend([bs_rows] * 4)
    args.append(dinv)
    in_specs.append(pl.BlockSpec((_RB, 1), lambda i: (i, 0)))
    args.extend([bst, ast, wst])
    in_specs.extend([
        pl.BlockSpec((4, D), lambda i: (0, 0)),
        pl.BlockSpec((4, D), lambda i: (0, 0)),
        pl.BlockSpec((4, D, D), lambda i: (0, 0, 0)),
    ])
    return pl.pallas_call(
        _tc_b_body,
        grid=(_GRID,),
        in_specs=in_specs,
        out_specs=[bs_rows] * 4,
        out_shape=[jax.ShapeDtypeStruct((NP, D), jnp.float32)
                   for _ in range(4)],
    )(*args)


def _tc_c_body(p0a, p0b, p1a, p1b, p2a, p2b, p3a, p3b,
               h0, h1, h2, h3, dinv, bst, ast,
               o0, o1, o2, o3, csum):
    i = pl.program_id(0)
    dv = dinv[...]
    hs = (h0, h1, h2, h3)
    pa = (p0a, p1a, p2a, p3a)
    pb = (p0b, p1b, p2b, p3b)
    outs = (o0, o1, o2, o3)
    rid = lax.broadcasted_iota(jnp.int32, (_RB, 1), 0) + i * _RB
    mask = rid < N
    zs = []
    for s in range(4):
        agg = dv * (pa[s][...] + pb[s][...] + hs[s][...]) + bst[s:s + 1, :]
        z = _prelu(agg, ast[s:s + 1, :])
        outs[s][...] = z
        zs.append(z)

    @pl.when(i == 0)
    def _():
        csum[...] = jnp.zeros_like(csum)

    c0 = jnp.sum(jnp.where(mask, zs[0], 0.0), axis=0, keepdims=True)
    c1 = jnp.sum(jnp.where(mask, zs[1], 0.0), axis=0, keepdims=True)
    csum[...] = csum[...] + jnp.concatenate([c0, c1], axis=0)


def _tc_c(parts, hs, dinv, bst, ast):
    bs_part = [pl.BlockSpec((_RB, D), lambda i, o=o: (i + o * _GRID, 0))
               for o in (0, 1)]
    bs_rows = pl.BlockSpec((_RB, D), lambda i: (i, 0))
    in_specs = []
    args = []
    for p in parts:
        args.extend([p, p])
        in_specs.extend(bs_part)
    args.extend(hs)
    in_specs.extend([bs_rows] * 4)
    args.append(dinv)
    in_specs.append(pl.BlockSpec((_RB, 1), lambda i: (i, 0)))
    args.extend([bst, ast])
    in_specs.extend([
        pl.BlockSpec((4, D), lambda i: (0, 0)),
        pl.BlockSpec((4, D), lambda i: (0, 0)),
    ])
    out_shape = [jax.ShapeDtypeStruct((NP, D), jnp.float32)
                 for _ in range(4)]
    out_shape.append(jax.ShapeDtypeStruct((2, D), jnp.float32))
    out_specs = [bs_rows] * 4 + [pl.BlockSpec((2, D), lambda i: (0, 0))]
    return pl.pallas_call(
        _tc_c_body,
        grid=(_GRID,),
        in_specs=in_specs,
        out_specs=out_specs,
        out_shape=out_shape,
    )(*args)


def _tc_d_body(csum, wp, bp, g):
    m = jax.nn.sigmoid(csum[...] * (1.0 / N))
    g[...] = jnp.dot(m, wp[...], preferred_element_type=jnp.float32) + bp[...]


def _tc_d(csum, wp, bp):
    return pl.pallas_call(
        _tc_d_body,
        out_shape=jax.ShapeDtypeStruct((2, D), jnp.float32),
    )(csum, wp, bp)


def kernel(x, edge_index, edge_weight, W1a, b1a, W2a, b2a, a1,
           W1b, b1b, W2b, b2b, a2, Wp, bp, perm1, perm2):
    src = edge_index[0].astype(jnp.int32)
    dst = edge_index[1].astype(jnp.int32)
    w = edge_weight.astype(jnp.float32)
    p1 = perm1.astype(jnp.int32)
    p2 = perm2.astype(jnp.int32)

    degflat, xp1, xp2 = _sc_pre(dst, w, x, p1, p2)
    deg2 = degflat.reshape(NC, NP)
    d0 = deg2[0].reshape(NP, 1)
    d1 = deg2[1].reshape(NP, 1)

    pad = jnp.zeros((NP - N, D), jnp.float32)
    xpd = jnp.concatenate([x, pad], axis=0)
    xp1d = jnp.concatenate([xp1, pad], axis=0)
    xp2d = jnp.concatenate([xp2, pad], axis=0)

    hs0, hs1, hs2, hs3, dinv = _tc_a(d0, d1, xpd, xp1d, xp2d, W1a, W1b)

    parts1 = _sc_mp(src, dst, w, hs0, hs1, hs2, hs3)

    bst1 = jnp.stack([b1a, b1b, b1a, b1b])
    bst2 = jnp.stack([b2a, b2b, b2a, b2b])
    ast = jnp.stack([a1, a2, a1, a2])
    wst2 = jnp.stack([W2a, W2b, W2a, W2b])

    g0, g1_, g2_, g3_ = _tc_b(parts1, (hs0, hs1, hs2, hs3), dinv,
                              bst1, ast, wst2)

    parts2 = _sc_mp(src, dst, w, g0, g1_, g2_, g3_)

    z0, z1_, z2_, z3_, csum = _tc_c(parts2, (g0, g1_, g2_, g3_), dinv,
                                    bst2, ast)

    g12 = _tc_d(csum, Wp, bp.reshape(1, D))

    z1 = z0[:N]
    z2 = z1_[:N]
    z1n = z2_[:N]
    z2n = z3_[:N]
    return (z1, z2, g12[0:1], g12[1:2], z1n, z2n,
            jnp.arange(N, dtype=jnp.int32), x.shape[0])


# speedup vs baseline: 5.5511x; 5.5511x over previous
import functools

import jax
import jax.numpy as jnp
from jax import lax
from jax.experimental import pallas as pl
from jax.experimental.pallas import tpu as pltpu
from jax.experimental.pallas import tpu_sc as plsc

N = 10000
NP = 10240
E = 320000
D = 128
NC = 2
NS = 16
CH = 80
ECHUNKS = E // (NC * NS) // CH
GCHUNKS = N // CH

_MESH = plsc.VectorSubcoreMesh(
    core_axis_name="c", subcore_axis_name="s", num_cores=NC, num_subcores=NS)


def _zero_vmem_rows(buf, nrows):
    z16 = jnp.zeros((16,), jnp.float32)

    def body(i, _):
        for j in range(D // 16):
            buf[i, pl.ds(j * 16, 16)] = z16
        return 0

    lax.fori_loop(0, nrows, body, 0)


@functools.partial(
    pl.kernel,
    out_type=[
        jax.ShapeDtypeStruct((NC * NP,), jnp.float32),
        jax.ShapeDtypeStruct((N, D), jnp.float32),
        jax.ShapeDtypeStruct((N, D), jnp.float32),
    ],
    mesh=_MESH,
    scratch_types=[
        pltpu.VMEM((CH,), jnp.int32),
        pltpu.VMEM((CH,), jnp.float32),
        pltpu.VMEM((CH, D), jnp.float32),
        pltpu.VMEM_SHARED((NP,), jnp.float32),
    ],
)
def _sc_pre(dst_hbm, w_hbm, x_hbm, p1_hbm, p2_hbm,
            deg_hbm, xp1_hbm, xp2_hbm,
            idxb, valb, rows, dacc):
    cid = lax.axis_index("c")
    sid = lax.axis_index("s")
    wid = sid * NC + cid
    zslice = NP // NS

    valb[...] = jnp.zeros((CH,), jnp.float32)

    def zbody(i, _):
        pltpu.sync_copy(valb, dacc.at[pl.ds(sid * zslice + i * CH, CH)])
        return 0

    lax.fori_loop(0, zslice // CH, zbody, 0)
    plsc.subcore_barrier()

    base_e = (cid * NS + sid) * (ECHUNKS * CH)

    def ebody(c, _):
        off = base_e + c * CH
        pltpu.sync_copy(dst_hbm.at[pl.ds(off, CH)], idxb)
        pltpu.sync_copy(w_hbm.at[pl.ds(off, CH)], valb)
        pltpu.sync_copy(valb, dacc.at[idxb], add=True)
        return 0

    lax.fori_loop(0, ECHUNKS, ebody, 0)
    plsc.subcore_barrier()

    pltpu.sync_copy(dacc.at[pl.ds(sid * zslice, zslice)],
                    deg_hbm.at[pl.ds(cid * NP + sid * zslice, zslice)])

    for i in range((GCHUNKS + NC * NS - 1) // (NC * NS)):
        c = wid + i * NC * NS

        @pl.when(c < GCHUNKS)
        def _():
            off = c * CH
            pltpu.sync_copy(p1_hbm.at[pl.ds(off, CH)], idxb)
            pltpu.sync_copy(x_hbm.at[idxb], rows)
            pltpu.sync_copy(rows, xp1_hbm.at[pl.ds(off, CH)])
            pltpu.sync_copy(p2_hbm.at[pl.ds(off, CH)], idxb)
            pltpu.sync_copy(x_hbm.at[idxb], rows)
            pltpu.sync_copy(rows, xp2_hbm.at[pl.ds(off, CH)])


@functools.partial(
    pl.kernel,
    out_type=[jax.ShapeDtypeStruct((NC * NP, D), jnp.float32)
              for _ in range(4)],
    mesh=_MESH,
    scratch_types=[
        pltpu.VMEM((CH,), jnp.int32),
        pltpu.VMEM((CH,), jnp.int32),
        pltpu.VMEM((CH,), jnp.float32),
        pltpu.VMEM((CH, D), jnp.float32),
        pltpu.VMEM((CH, D), jnp.float32),
        pltpu.VMEM_SHARED((NP, D), jnp.float32),
    ],
)
def _sc_mp(src_hbm, dst_hbm, w_hbm, hs0, hs1, hs2, hs3,
           part0, part1, part2, part3,
           idx_s, idx_d, wv, rows, zrows, acc):
    cid = lax.axis_index("c")
    sid = lax.axis_index("s")
    zslice = NP // NS

    _zero_vmem_rows(zrows, CH)

    hs_list = (hs0, hs1, hs2, hs3)
    part_list = (part0, part1, part2, part3)

    for s in range(4):
        hs = hs_list[s]
        part = part_list[s]

        for j in range(zslice // CH):
            pltpu.sync_copy(zrows, acc.at[pl.ds(sid * zslice + j * CH, CH)])
        plsc.subcore_barrier()

        base_e = (cid * NS + sid) * (ECHUNKS * CH)

        def ebody(c, _):
            off = base_e + c * CH
            pltpu.sync_copy(src_hbm.at[pl.ds(off, CH)], idx_s)
            pltpu.sync_copy(dst_hbm.at[pl.ds(off, CH)], idx_d)
            pltpu.sync_copy(w_hbm.at[pl.ds(off, CH)], wv)
            pltpu.sync_copy(hs.at[idx_s], rows)

            def sbody(t, _2):
                wvec = wv[pl.ds(t * 16, 16)]
                for kk in range(16):
                    svec = jnp.full((16,), wvec[kk], jnp.float32)
                    k = t * 16 + kk
                    for j in range(D // 16):
                        sl = pl.ds(j * 16, 16)
                        rows[k, sl] = rows[k, sl] * svec
                return 0

            lax.fori_loop(0, CH // 16, sbody, 0)
            pltpu.sync_copy(rows, acc.at[idx_d], add=True)
            return 0

        lax.fori_loop(0, ECHUNKS, ebody, 0)
        plsc.subcore_barrier()

        for j in range(zslice // CH):
            roff = sid * zslice + j * CH
            pltpu.sync_copy(acc.at[pl.ds(roff, CH)],
                            part.at[pl.ds(cid * NP + roff, CH)])
        plsc.subcore_barrier()


def _prelu(z, a):
    return jnp.maximum(z, 0.0) + a * jnp.minimum(z, 0.0)


def _tc_a_body(d0, d1, x, xp1, xp2, w1a, w1b,
               hs0, hs1, hs2, hs3, dinv):
    dv = lax.rsqrt(d0[...] + d1[...] + 1.0)
    dinv[...] = dv
    hs0[...] = jnp.dot(x[...], w1a[...], preferred_element_type=jnp.float32) * dv
    hs1[...] = jnp.dot(x[...], w1b[...], preferred_element_type=jnp.float32) * dv
    hs2[...] = jnp.dot(xp1[...], w1a[...], preferred_element_type=jnp.float32) * dv
    hs3[...] = jnp.dot(xp2[...], w1b[...], preferred_element_type=jnp.float32) * dv


def _tc_a(d0, d1, x, xp1, xp2, w1a, w1b):
    outs = [jax.ShapeDtypeStruct((NP, D), jnp.float32) for _ in range(4)]
    outs.append(jax.ShapeDtypeStruct((NP, 1), jnp.float32))
    return pl.pallas_call(
        _tc_a_body,
        out_shape=outs,
    )(d0, d1, x, xp1, xp2, w1a, w1b)


_RB = 2048
_GRID = NP // _RB


def _tc_b_body(p0a, p0b, p1a, p1b, p2a, p2b, p3a, p3b,
               h0, h1, h2, h3, dinv, bst, ast, wst,
               o0, o1, o2, o3):
    dv = dinv[...]
    hs = (h0, h1, h2, h3)
    pa = (p0a, p1a, p2a, p3a)
    pb = (p0b, p1b, p2b, p3b)
    outs = (o0, o1, o2, o3)
    for s in range(4):
        agg = dv * (pa[s][...] + pb[s][...] + hs[s][...]) + bst[s:s + 1, :]
        z = _prelu(agg, ast[s:s + 1, :])
        outs[s][...] = jnp.dot(z, wst[s],
                               preferred_element_type=jnp.float32) * dv


def _tc_b(parts, hs, dinv, bst, ast, wst):
    bs_part = [pl.BlockSpec((_RB, D), lambda i, o=o: (i + o * _GRID, 0))
               for o in (0, 1)]
    bs_rows = pl.BlockSpec((_RB, D), lambda i: (i, 0))
    in_specs = []
    args = []
    for p in parts:
        args.extend([p, p])
        in_specs.extend(bs_part)
    args.extend(hs)
    in_specs.extend([bs_rows] * 4)
    args.append(dinv)
    in_specs.append(pl.BlockSpec((_RB, 1), lambda i: (i, 0)))
    args.extend([bst, ast, wst])
    in_specs.extend([
        pl.BlockSpec((4, D), lambda i: (0, 0)),
        pl.BlockSpec((4, D), lambda i: (0, 0)),
        pl.BlockSpec((4, D, D), lambda i: (0, 0, 0)),
    ])
    return pl.pallas_call(
        _tc_b_body,
        grid=(_GRID,),
        in_specs=in_specs,
        out_specs=[bs_rows] * 4,
        out_shape=[jax.ShapeDtypeStruct((NP, D), jnp.float32)
                   for _ in range(4)],
    )(*args)


def _tc_c_body(p0a, p0b, p1a, p1b, p2a, p2b, p3a, p3b,
               h0, h1, h2, h3, dinv, bst, ast,
               o0, o1, o2, o3, csum):
    i = pl.program_id(0)
    dv = dinv[...]
    hs = (h0, h1, h2, h3)
    pa = (p0a, p1a, p2a, p3a)
    pb = (p0b, p1b, p2b, p3b)
    outs = (o0, o1, o2, o3)
    rid = lax.broadcasted_iota(jnp.int32, (_RB, 1), 0) + i * _RB
    mask = rid < N
    zs = []
    for s in range(4):
        agg = dv * (pa[s][...] + pb[s][...] + hs[s][...]) + bst[s:s + 1, :]
        z = _prelu(agg, ast[s:s + 1, :])
        outs[s][...] = z
        zs.append(z)

    @pl.when(i == 0)
    def _():
        csum[...] = jnp.zeros_like(csum)

    c0 = jnp.sum(jnp.where(mask, zs[0], 0.0), axis=0, keepdims=True)
    c1 = jnp.sum(jnp.where(mask, zs[1], 0.0), axis=0, keepdims=True)
    csum[...] = csum[...] + jnp.concatenate([c0, c1], axis=0)


def _tc_c(parts, hs, dinv, bst, ast):
    bs_part = [pl.BlockSpec((_RB, D), lambda i, o=o: (i + o * _GRID, 0))
               for o in (0, 1)]
    bs_rows = pl.BlockSpec((_RB, D), lambda i: (i, 0))
    in_specs = []
    args = []
    for p in parts:
        args.extend([p, p])
        in_specs.extend(bs_part)
    args.extend(hs)
    in_specs.extend([bs_rows] * 4)
    args.append(dinv)
    in_specs.append(pl.BlockSpec((_RB, 1), lambda i: (i, 0)))
    args.extend([bst, ast])
    in_specs.extend([
        pl.BlockSpec((4, D), lambda i: (0, 0)),
        pl.BlockSpec((4, D), lambda i: (0, 0)),
    ])
    out_shape = [jax.ShapeDtypeStruct((NP, D), jnp.float32)
                 for _ in range(4)]
    out_shape.append(jax.ShapeDtypeStruct((2, D), jnp.float32))
    out_specs = [bs_rows] * 4 + [pl.BlockSpec((2, D), lambda i: (0, 0))]
    return pl.pallas_call(
        _tc_c_body,
        grid=(_GRID,),
        in_specs=in_specs,
        out_specs=out_specs,
        out_shape=out_shape,
    )(*args)


def _tc_d_body(csum, wp, bp, g):
    m = jax.nn.sigmoid(csum[...] * (1.0 / N))
    g[...] = jnp.dot(m, wp[...], preferred_element_type=jnp.float32) + bp[...]


def _tc_d(csum, wp, bp):
    return pl.pallas_call(
        _tc_d_body,
        out_shape=jax.ShapeDtypeStruct((2, D), jnp.float32),
    )(csum, wp, bp)


def kernel(x, edge_index, edge_weight, W1a, b1a, W2a, b2a, a1,
           W1b, b1b, W2b, b2b, a2, Wp, bp, perm1, perm2):
    src = edge_index[0].astype(jnp.int32)
    dst = edge_index[1].astype(jnp.int32)
    w = edge_weight.astype(jnp.float32)
    p1 = perm1.astype(jnp.int32)
    p2 = perm2.astype(jnp.int32)

    degflat, xp1, xp2 = _sc_pre(dst, w, x, p1, p2)
    deg2 = degflat.reshape(NC, NP)
    d0 = deg2[0].reshape(NP, 1)
    d1 = deg2[1].reshape(NP, 1)

    pad = jnp.zeros((NP - N, D), jnp.float32)
    xpd = jnp.concatenate([x, pad], axis=0)
    xp1d = jnp.concatenate([xp1, pad], axis=0)
    xp2d = jnp.concatenate([xp2, pad], axis=0)

    hs0, hs1, hs2, hs3, dinv = _tc_a(d0, d1, xpd, xp1d, xp2d, W1a, W1b)

    parts1 = _sc_mp(src, dst, w, hs0, hs1, hs2, hs3)

    bst1 = jnp.stack([b1a, b1b, b1a, b1b])
    bst2 = jnp.stack([b2a, b2b, b2a, b2b])
    ast = jnp.stack([a1, a2, a1, a2])
    wst2 = jnp.stack([W2a, W2b, W2a, W2b])

    g0, g1_, g2_, g3_ = _tc_b(parts1, (hs0, hs1, hs2, hs3), dinv,
                              bst1, ast, wst2)

    parts2 = _sc_mp(src, dst, w, g0, g1_, g2_, g3_)

    z0, z1_, z2_, z3_, csum = _tc_c(parts2, (g0, g1_, g2_, g3_), dinv,
                                    bst2, ast)

    g12 = _tc_d(csum, Wp, bp.reshape(1, D))

    z1 = z0[:N]
    z2 = z1_[:N]
    z1n = z2_[:N]
    z2n = z3_[:N]
    return (z1, z2, g12[0:1], g12[1:2], z1n, z2n,
            jnp.arange(N, dtype=jnp.int32), x.shape[0])


# speedup vs baseline: 9.9650x; 1.7951x over previous
import functools

import jax
import jax.numpy as jnp
from jax import lax
from jax.experimental import pallas as pl
from jax.experimental.pallas import tpu as pltpu
from jax.experimental.pallas import tpu_sc as plsc

N = 10000
NP = 10240
E = 320000
D = 128
NC = 2
NS = 16
CH = 80
ECHUNKS = E // (NC * NS) // CH
GCHUNKS = N // CH
NB = 5
GROUPS = ECHUNKS // NB

_MESH = plsc.VectorSubcoreMesh(
    core_axis_name="c", subcore_axis_name="s", num_cores=NC, num_subcores=NS)


def _zero_vmem_rows(buf, nrows):
    z16 = jnp.zeros((16,), jnp.float32)

    def body(i, _):
        for j in range(D // 16):
            buf[i, pl.ds(j * 16, 16)] = z16
        return 0

    lax.fori_loop(0, nrows, body, 0)


@functools.partial(
    pl.kernel,
    out_type=[
        jax.ShapeDtypeStruct((NC * NP,), jnp.float32),
        jax.ShapeDtypeStruct((N, D), jnp.float32),
        jax.ShapeDtypeStruct((N, D), jnp.float32),
    ],
    mesh=_MESH,
    scratch_types=[
        pltpu.VMEM((CH,), jnp.int32),
        pltpu.VMEM((CH,), jnp.float32),
        pltpu.VMEM((CH, D), jnp.float32),
        pltpu.VMEM_SHARED((NP,), jnp.float32),
    ],
)
def _sc_pre(dst_hbm, w_hbm, x_hbm, p1_hbm, p2_hbm,
            deg_hbm, xp1_hbm, xp2_hbm,
            idxb, valb, rows, dacc):
    cid = lax.axis_index("c")
    sid = lax.axis_index("s")
    wid = sid * NC + cid
    zslice = NP // NS

    valb[...] = jnp.zeros((CH,), jnp.float32)

    def zbody(i, _):
        pltpu.sync_copy(valb, dacc.at[pl.ds(sid * zslice + i * CH, CH)])
        return 0

    lax.fori_loop(0, zslice // CH, zbody, 0)
    plsc.subcore_barrier()

    base_e = (cid * NS + sid) * (ECHUNKS * CH)

    def ebody(c, _):
        off = base_e + c * CH
        pltpu.sync_copy(dst_hbm.at[pl.ds(off, CH)], idxb)
        pltpu.sync_copy(w_hbm.at[pl.ds(off, CH)], valb)
        pltpu.sync_copy(valb, dacc.at[idxb], add=True)
        return 0

    lax.fori_loop(0, ECHUNKS, ebody, 0)
    plsc.subcore_barrier()

    pltpu.sync_copy(dacc.at[pl.ds(sid * zslice, zslice)],
                    deg_hbm.at[pl.ds(cid * NP + sid * zslice, zslice)])

    for i in range((GCHUNKS + NC * NS - 1) // (NC * NS)):
        c = wid + i * NC * NS

        @pl.when(c < GCHUNKS)
        def _():
            off = c * CH
            pltpu.sync_copy(p1_hbm.at[pl.ds(off, CH)], idxb)
            pltpu.sync_copy(x_hbm.at[idxb], rows)
            pltpu.sync_copy(rows, xp1_hbm.at[pl.ds(off, CH)])
            pltpu.sync_copy(p2_hbm.at[pl.ds(off, CH)], idxb)
            pltpu.sync_copy(x_hbm.at[idxb], rows)
            pltpu.sync_copy(rows, xp2_hbm.at[pl.ds(off, CH)])


@functools.partial(
    pl.kernel,
    out_type=jax.ShapeDtypeStruct((4 * NC * NP, D), jnp.float32),
    mesh=_MESH,
    scratch_types=[
        [pltpu.VMEM((CH,), jnp.int32) for _ in range(2)],
        [pltpu.VMEM((CH,), jnp.int32) for _ in range(2)],
        [pltpu.VMEM((CH,), jnp.float32) for _ in range(2)],
        [pltpu.VMEM((CH, D), jnp.float32) for _ in range(2)],
        pltpu.VMEM_SHARED((NP, D), jnp.float32),
        [pltpu.SemaphoreType.DMA for _ in range(2)],
        [pltpu.SemaphoreType.DMA for _ in range(2)],
        pltpu.SemaphoreType.DMA,
    ],
)
def _sc_mp(src_hbm, dst_hbm, w_hbm, hs_hbm, part_hbm,
           srcb, dstb, wb, rows, acc, isems, gsems, ssem):
    cid = lax.axis_index("c")
    sid = lax.axis_index("s")
    zslice = NP // NS
    base_e = (cid * NS + sid) * (ECHUNKS * CH)

    def _scale(rows_k, wb_k):
        def sbody(t, _2):
            wvec = wb_k[pl.ds(t * 16, 16)]
            for kk in range(16):
                svec = jnp.full((16,), wvec[kk], jnp.float32)
                r = t * 16 + kk
                for j in range(D // 16):
                    sl = pl.ds(j * 16, 16)
                    rows_k[r, sl] = rows_k[r, sl] * svec
            return 0

        lax.fori_loop(0, CH // 16, sbody, 0)

    def stream_body(s, _):
        _zero_vmem_rows(rows[0], CH)
        for j in range(zslice // CH):
            pltpu.sync_copy(rows[0], acc.at[pl.ds(sid * zslice + j * CH, CH)])
        plsc.subcore_barrier()

        soff = jnp.full((16,), s * NP, jnp.int32)

        def fetch_idx(c, k):
            off = base_e + c * CH
            return [pltpu.async_copy(src_hbm.at[pl.ds(off, CH)], srcb[k],
                                     isems[k]),
                    pltpu.async_copy(dst_hbm.at[pl.ds(off, CH)], dstb[k],
                                     isems[k]),
                    pltpu.async_copy(w_hbm.at[pl.ds(off, CH)], wb[k],
                                     isems[k])]

        def fire_gather(k):
            for j in range(CH // 16):
                sl = pl.ds(j * 16, 16)
                srcb[k][sl] = srcb[k][sl] + soff
            return pltpu.async_copy(hs_hbm.at[srcb[k]], rows[k], gsems[k])

        def gbody(g, _g):
            c0 = g * 2
            c1 = c0 + 1
            i0 = fetch_idx(c0, 0)
            i1 = fetch_idx(c1, 1)
            for d in i0:
                d.wait()
            g0 = fire_gather(0)
            for d in i1:
                d.wait()
            g1 = fire_gather(1)
            g0.wait()
            _scale(rows[0], wb[0])
            s0 = pltpu.async_copy(rows[0], acc.at[dstb[0]], ssem, add=True)
            g1.wait()
            _scale(rows[1], wb[1])
            s1 = pltpu.async_copy(rows[1], acc.at[dstb[1]], ssem, add=True)
            s0.wait()
            s1.wait()
            return 0

        lax.fori_loop(0, ECHUNKS // 2, gbody, 0)

        it = fetch_idx(ECHUNKS - 1, 0)
        for d in it:
            d.wait()
        gt = fire_gather(0)
        gt.wait()
        _scale(rows[0], wb[0])
        st = pltpu.async_copy(rows[0], acc.at[dstb[0]], ssem, add=True)
        st.wait()
        plsc.subcore_barrier()

        for j in range(zslice // CH):
            roff = sid * zslice + j * CH
            pltpu.sync_copy(
                acc.at[pl.ds(roff, CH)],
                part_hbm.at[pl.ds(s * (NC * NP) + cid * NP + roff, CH)])
        plsc.subcore_barrier()
        return 0

    lax.fori_loop(0, 4, stream_body, 0)


def _prelu(z, a):
    return jnp.maximum(z, 0.0) + a * jnp.minimum(z, 0.0)


def _tc_a_body(d0, d1, x, xp1, xp2, w1a, w1b, hs, dinv):
    dv = lax.rsqrt(d0[...] + d1[...] + 1.0)
    dinv[...] = dv
    hs[0:NP, :] = jnp.dot(x[...], w1a[...],
                          preferred_element_type=jnp.float32) * dv
    hs[NP:2 * NP, :] = jnp.dot(x[...], w1b[...],
                               preferred_element_type=jnp.float32) * dv
    hs[2 * NP:3 * NP, :] = jnp.dot(xp1[...], w1a[...],
                                   preferred_element_type=jnp.float32) * dv
    hs[3 * NP:4 * NP, :] = jnp.dot(xp2[...], w1b[...],
                                   preferred_element_type=jnp.float32) * dv


def _tc_a(d0, d1, x, xp1, xp2, w1a, w1b):
    return pl.pallas_call(
        _tc_a_body,
        out_shape=[jax.ShapeDtypeStruct((4 * NP, D), jnp.float32),
                   jax.ShapeDtypeStruct((NP, 1), jnp.float32)],
    )(d0, d1, x, xp1, xp2, w1a, w1b)


_RB = 2048
_RG = NP // _RB


def _tc_b_body(pa, pb, h, dinv, bst, ast, wst, o):
    s = pl.program_id(0)
    dv = dinv[...]
    agg = dv * (pa[...] + pb[...] + h[...]) + bst[pl.ds(s, 1), :]
    z = _prelu(agg, ast[pl.ds(s, 1), :])
    wmat = wst[pl.ds(s, 1), :, :][0]
    o[...] = jnp.dot(z, wmat, preferred_element_type=jnp.float32) * dv


def _tc_b(parts, hs, dinv, bst, ast, wst):
    return pl.pallas_call(
        _tc_b_body,
        grid=(4, _RG),
        in_specs=[
            pl.BlockSpec((_RB, D), lambda s, i: (2 * _RG * s + i, 0)),
            pl.BlockSpec((_RB, D), lambda s, i: (2 * _RG * s + _RG + i, 0)),
            pl.BlockSpec((_RB, D), lambda s, i: (_RG * s + i, 0)),
            pl.BlockSpec((_RB, 1), lambda s, i: (i, 0)),
            pl.BlockSpec((4, D), lambda s, i: (0, 0)),
            pl.BlockSpec((4, D), lambda s, i: (0, 0)),
            pl.BlockSpec((4, D, D), lambda s, i: (0, 0, 0)),
        ],
        out_specs=pl.BlockSpec((_RB, D), lambda s, i: (_RG * s + i, 0)),
        out_shape=jax.ShapeDtypeStruct((4 * NP, D), jnp.float32),
    )(parts, parts, hs, dinv, bst, ast, wst)


def _tc_c_body(pa, pb, h, dinv, bst, ast, o, csum):
    s = pl.program_id(0)
    i = pl.program_id(1)
    dv = dinv[...]
    agg = dv * (pa[...] + pb[...] + h[...]) + bst[pl.ds(s, 1), :]
    z = _prelu(agg, ast[pl.ds(s, 1), :])
    o[...] = z

    @pl.when(jnp.logical_and(s == 0, i == 0))
    def _():
        csum[...] = jnp.zeros_like(csum)

    rid = lax.broadcasted_iota(jnp.int32, (_RB, 1), 0) + i * _RB
    zm = jnp.where(rid < N, z, 0.0)
    csum[pl.ds(s, 1), :] = (csum[pl.ds(s, 1), :]
                            + jnp.sum(zm, axis=0, keepdims=True))


def _tc_c(parts, hs, dinv, bst, ast):
    return pl.pallas_call(
        _tc_c_body,
        grid=(4, _RG),
        in_specs=[
            pl.BlockSpec((_RB, D), lambda s, i: (2 * _RG * s + i, 0)),
            pl.BlockSpec((_RB, D), lambda s, i: (2 * _RG * s + _RG + i, 0)),
            pl.BlockSpec((_RB, D), lambda s, i: (_RG * s + i, 0)),
            pl.BlockSpec((_RB, 1), lambda s, i: (i, 0)),
            pl.BlockSpec((4, D), lambda s, i: (0, 0)),
            pl.BlockSpec((4, D), lambda s, i: (0, 0)),
        ],
        out_specs=[pl.BlockSpec((_RB, D), lambda s, i: (_RG * s + i, 0)),
                   pl.BlockSpec((4, D), lambda s, i: (0, 0))],
        out_shape=[jax.ShapeDtypeStruct((4 * NP, D), jnp.float32),
                   jax.ShapeDtypeStruct((4, D), jnp.float32)],
    )(parts, parts, hs, dinv, bst, ast)


def _tc_d_body(csum, wp, bp, g):
    m = jax.nn.sigmoid(csum[...] * (1.0 / N))
    g[...] = jnp.dot(m, wp[...], preferred_element_type=jnp.float32) + bp[...]


def _tc_d(csum, wp, bp):
    return pl.pallas_call(
        _tc_d_body,
        out_shape=jax.ShapeDtypeStruct((2, D), jnp.float32),
    )(csum, wp, bp)


def kernel(x, edge_index, edge_weight, W1a, b1a, W2a, b2a, a1,
           W1b, b1b, W2b, b2b, a2, Wp, bp, perm1, perm2):
    src = edge_index[0].astype(jnp.int32)
    dst = edge_index[1].astype(jnp.int32)
    w = edge_weight.astype(jnp.float32)
    p1 = perm1.astype(jnp.int32)
    p2 = perm2.astype(jnp.int32)

    degflat, xp1, xp2 = _sc_pre(dst, w, x, p1, p2)
    deg2 = degflat.reshape(NC, NP)
    d0 = deg2[0].reshape(NP, 1)
    d1 = deg2[1].reshape(NP, 1)

    pad = jnp.zeros((NP - N, D), jnp.float32)
    xpd = jnp.concatenate([x, pad], axis=0)
    xp1d = jnp.concatenate([xp1, pad], axis=0)
    xp2d = jnp.concatenate([xp2, pad], axis=0)

    hs, dinv = _tc_a(d0, d1, xpd, xp1d, xp2d, W1a, W1b)

    parts1 = _sc_mp(src, dst, w, hs)

    bst1 = jnp.stack([b1a, b1b, b1a, b1b])
    bst2 = jnp.stack([b2a, b2b, b2a, b2b])
    ast = jnp.stack([a1, a2, a1, a2])
    wst2 = jnp.stack([W2a, W2b, W2a, W2b])

    hs2 = _tc_b(parts1, hs, dinv, bst1, ast, wst2)

    parts2 = _sc_mp(src, dst, w, hs2)

    zall, csum = _tc_c(parts2, hs2, dinv, bst2, ast)

    g12 = _tc_d(csum[0:2], Wp, bp.reshape(1, D))

    z1 = zall[0:N]
    z2 = zall[NP:NP + N]
    z1n = zall[2 * NP:2 * NP + N]
    z2n = zall[3 * NP:3 * NP + N]
    return (z1, z2, g12[0:1], g12[1:2], z1n, z2n,
            jnp.arange(N, dtype=jnp.int32), x.shape[0])


# speedup vs baseline: 11.5423x; 1.1583x over previous
import functools

import jax
import jax.numpy as jnp
from jax import lax
from jax.experimental import pallas as pl
from jax.experimental.pallas import tpu as pltpu
from jax.experimental.pallas import tpu_sc as plsc

N = 10000
NP = 10240
E = 320000
D = 128
NC = 2
NS = 16
CH = 80
ECHUNKS = E // (NC * NS) // CH
GCHUNKS = N // CH
NB = 5
GROUPS = ECHUNKS // NB

_MESH = plsc.VectorSubcoreMesh(
    core_axis_name="c", subcore_axis_name="s", num_cores=NC, num_subcores=NS)


def _zero_vmem_rows(buf, nrows):
    z16 = jnp.zeros((16,), jnp.float32)

    def body(i, _):
        for j in range(D // 16):
            buf[i, pl.ds(j * 16, 16)] = z16
        return 0

    lax.fori_loop(0, nrows, body, 0)


@functools.partial(
    pl.kernel,
    out_type=[
        jax.ShapeDtypeStruct((NC * NP,), jnp.float32),
        jax.ShapeDtypeStruct((N, D), jnp.float32),
        jax.ShapeDtypeStruct((N, D), jnp.float32),
    ],
    mesh=_MESH,
    scratch_types=[
        pltpu.VMEM((CH,), jnp.int32),
        pltpu.VMEM((CH,), jnp.float32),
        pltpu.VMEM((CH, D), jnp.float32),
        pltpu.VMEM_SHARED((NP,), jnp.float32),
    ],
)
def _sc_pre(dst_hbm, w_hbm, x_hbm, p1_hbm, p2_hbm,
            deg_hbm, xp1_hbm, xp2_hbm,
            idxb, valb, rows, dacc):
    cid = lax.axis_index("c")
    sid = lax.axis_index("s")
    wid = sid * NC + cid
    zslice = NP // NS

    valb[...] = jnp.zeros((CH,), jnp.float32)

    def zbody(i, _):
        pltpu.sync_copy(valb, dacc.at[pl.ds(sid * zslice + i * CH, CH)])
        return 0

    lax.fori_loop(0, zslice // CH, zbody, 0)
    plsc.subcore_barrier()

    base_e = (cid * NS + sid) * (ECHUNKS * CH)

    def ebody(c, _):
        off = base_e + c * CH
        pltpu.sync_copy(dst_hbm.at[pl.ds(off, CH)], idxb)
        pltpu.sync_copy(w_hbm.at[pl.ds(off, CH)], valb)
        pltpu.sync_copy(valb, dacc.at[idxb], add=True)
        return 0

    lax.fori_loop(0, ECHUNKS, ebody, 0)
    plsc.subcore_barrier()

    pltpu.sync_copy(dacc.at[pl.ds(sid * zslice, zslice)],
                    deg_hbm.at[pl.ds(cid * NP + sid * zslice, zslice)])

    for i in range((GCHUNKS + NC * NS - 1) // (NC * NS)):
        c = wid + i * NC * NS

        @pl.when(c < GCHUNKS)
        def _():
            off = c * CH
            pltpu.sync_copy(p1_hbm.at[pl.ds(off, CH)], idxb)
            pltpu.sync_copy(x_hbm.at[idxb], rows)
            pltpu.sync_copy(rows, xp1_hbm.at[pl.ds(off, CH)])
            pltpu.sync_copy(p2_hbm.at[pl.ds(off, CH)], idxb)
            pltpu.sync_copy(x_hbm.at[idxb], rows)
            pltpu.sync_copy(rows, xp2_hbm.at[pl.ds(off, CH)])


@functools.partial(
    pl.kernel,
    out_type=jax.ShapeDtypeStruct((4 * NC * NP, D), jnp.float32),
    mesh=_MESH,
    scratch_types=[
        [pltpu.VMEM((CH,), jnp.int32) for _ in range(2)],
        [pltpu.VMEM((CH,), jnp.int32) for _ in range(2)],
        [pltpu.VMEM((CH,), jnp.float32) for _ in range(2)],
        [pltpu.VMEM((CH, D), jnp.float32) for _ in range(2)],
        pltpu.VMEM_SHARED((NP, D), jnp.float32),
        [pltpu.SemaphoreType.DMA for _ in range(2)],
        [pltpu.SemaphoreType.DMA for _ in range(2)],
        pltpu.SemaphoreType.DMA,
    ],
)
def _sc_mp(src_hbm, dst_hbm, w_hbm, hs_hbm, part_hbm,
           srcb, dstb, wb, rows, acc, isems, gsems, ssem):
    cid = lax.axis_index("c")
    sid = lax.axis_index("s")
    zslice = NP // NS
    base_e = (cid * NS + sid) * (ECHUNKS * CH)

    def _scale(rows_k, wb_k):
        def sbody(t, _2):
            wvec = wb_k[pl.ds(t * 16, 16)]
            for kk in range(16):
                svec = jnp.full((16,), wvec[kk], jnp.float32)
                r = t * 16 + kk
                for j in range(D // 16):
                    sl = pl.ds(j * 16, 16)
                    rows_k[r, sl] = rows_k[r, sl] * svec
            return 0

        lax.fori_loop(0, CH // 16, sbody, 0)

    def fetch_idx(c, k):
        off = base_e + c * CH
        return [pltpu.async_copy(src_hbm.at[pl.ds(off, CH)], srcb[k],
                                 isems[k]),
                pltpu.async_copy(dst_hbm.at[pl.ds(off, CH)], dstb[k],
                                 isems[k]),
                pltpu.async_copy(w_hbm.at[pl.ds(off, CH)], wb[k],
                                 isems[k])]

    def wait_idx(k):
        pltpu.make_async_copy(src_hbm.at[pl.ds(0, CH)], srcb[k],
                              isems[k]).wait()
        pltpu.make_async_copy(dst_hbm.at[pl.ds(0, CH)], dstb[k],
                              isems[k]).wait()
        pltpu.make_async_copy(w_hbm.at[pl.ds(0, CH)], wb[k],
                              isems[k]).wait()

    def stream_body(s, _):
        _zero_vmem_rows(rows[0], CH)
        zds = [pltpu.async_copy(rows[0],
                                acc.at[pl.ds(sid * zslice + j * CH, CH)],
                                gsems[0])
               for j in range(zslice // CH)]
        for d in zds:
            d.wait()
        plsc.subcore_barrier()

        soff = jnp.full((16,), s * NP, jnp.int32)

        def fire_gather(k):
            for j in range(CH // 16):
                sl = pl.ds(j * 16, 16)
                srcb[k][sl] = srcb[k][sl] + soff
            return pltpu.async_copy(hs_hbm.at[srcb[k]], rows[k], gsems[k])

        fetch_idx(0, 0)
        fetch_idx(1, 1)

        def gbody(g, _g):
            c0 = g * 2
            wait_idx(0)
            g0 = fire_gather(0)
            wait_idx(1)
            g1 = fire_gather(1)
            g0.wait()
            _scale(rows[0], wb[0])
            s0 = pltpu.async_copy(rows[0], acc.at[dstb[0]], ssem, add=True)
            g1.wait()
            _scale(rows[1], wb[1])
            s1 = pltpu.async_copy(rows[1], acc.at[dstb[1]], ssem, add=True)
            s0.wait()
            fetch_idx(jnp.minimum(c0 + 2, ECHUNKS - 1), 0)
            s1.wait()
            fetch_idx(jnp.minimum(c0 + 3, ECHUNKS - 1), 1)
            return 0

        lax.fori_loop(0, ECHUNKS // 2, gbody, 0)

        wait_idx(0)
        gt = fire_gather(0)
        wait_idx(1)
        gt.wait()
        _scale(rows[0], wb[0])
        st = pltpu.async_copy(rows[0], acc.at[dstb[0]], ssem, add=True)
        st.wait()
        plsc.subcore_barrier()

        dds = []
        for j in range(zslice // CH):
            roff = sid * zslice + j * CH
            dds.append(pltpu.async_copy(
                acc.at[pl.ds(roff, CH)],
                part_hbm.at[pl.ds(s * (NC * NP) + cid * NP + roff, CH)],
                gsems[1]))
        for d in dds:
            d.wait()
        plsc.subcore_barrier()
        return 0

    lax.fori_loop(0, 4, stream_body, 0)


def _prelu(z, a):
    return jnp.maximum(z, 0.0) + a * jnp.minimum(z, 0.0)


def _tc_a_body(d0, d1, x, xp1, xp2, w1a, w1b, hs, dinv):
    dv = lax.rsqrt(d0[...] + d1[...] + 1.0)
    dinv[...] = dv
    hs[0:NP, :] = jnp.dot(x[...], w1a[...],
                          preferred_element_type=jnp.float32) * dv
    hs[NP:2 * NP, :] = jnp.dot(x[...], w1b[...],
                               preferred_element_type=jnp.float32) * dv
    hs[2 * NP:3 * NP, :] = jnp.dot(xp1[...], w1a[...],
                                   preferred_element_type=jnp.float32) * dv
    hs[3 * NP:4 * NP, :] = jnp.dot(xp2[...], w1b[...],
                                   preferred_element_type=jnp.float32) * dv


def _tc_a(d0, d1, x, xp1, xp2, w1a, w1b):
    return pl.pallas_call(
        _tc_a_body,
        out_shape=[jax.ShapeDtypeStruct((4 * NP, D), jnp.float32),
                   jax.ShapeDtypeStruct((NP, 1), jnp.float32)],
    )(d0, d1, x, xp1, xp2, w1a, w1b)


_RB = 2048
_RG = NP // _RB


def _tc_b_body(pa, pb, h, dinv, bst, ast, wst, o):
    s = pl.program_id(0)
    dv = dinv[...]
    agg = dv * (pa[...] + pb[...] + h[...]) + bst[pl.ds(s, 1), :]
    z = _prelu(agg, ast[pl.ds(s, 1), :])
    wmat = wst[pl.ds(s, 1), :, :][0]
    o[...] = jnp.dot(z, wmat, preferred_element_type=jnp.float32) * dv


def _tc_b(parts, hs, dinv, bst, ast, wst):
    return pl.pallas_call(
        _tc_b_body,
        grid=(4, _RG),
        in_specs=[
            pl.BlockSpec((_RB, D), lambda s, i: (2 * _RG * s + i, 0)),
            pl.BlockSpec((_RB, D), lambda s, i: (2 * _RG * s + _RG + i, 0)),
            pl.BlockSpec((_RB, D), lambda s, i: (_RG * s + i, 0)),
            pl.BlockSpec((_RB, 1), lambda s, i: (i, 0)),
            pl.BlockSpec((4, D), lambda s, i: (0, 0)),
            pl.BlockSpec((4, D), lambda s, i: (0, 0)),
            pl.BlockSpec((4, D, D), lambda s, i: (0, 0, 0)),
        ],
        out_specs=pl.BlockSpec((_RB, D), lambda s, i: (_RG * s + i, 0)),
        out_shape=jax.ShapeDtypeStruct((4 * NP, D), jnp.float32),
    )(parts, parts, hs, dinv, bst, ast, wst)


def _tc_c_body(pa, pb, h, dinv, bst, ast, o, csum):
    s = pl.program_id(0)
    i = pl.program_id(1)
    dv = dinv[...]
    agg = dv * (pa[...] + pb[...] + h[...]) + bst[pl.ds(s, 1), :]
    z = _prelu(agg, ast[pl.ds(s, 1), :])
    o[...] = z

    @pl.when(jnp.logical_and(s == 0, i == 0))
    def _():
        csum[...] = jnp.zeros_like(csum)

    rid = lax.broadcasted_iota(jnp.int32, (_RB, 1), 0) + i * _RB
    zm = jnp.where(rid < N, z, 0.0)
    csum[pl.ds(s, 1), :] = (csum[pl.ds(s, 1), :]
                            + jnp.sum(zm, axis=0, keepdims=True))


def _tc_c(parts, hs, dinv, bst, ast):
    return pl.pallas_call(
        _tc_c_body,
        grid=(4, _RG),
        in_specs=[
            pl.BlockSpec((_RB, D), lambda s, i: (2 * _RG * s + i, 0)),
            pl.BlockSpec((_RB, D), lambda s, i: (2 * _RG * s + _RG + i, 0)),
            pl.BlockSpec((_RB, D), lambda s, i: (_RG * s + i, 0)),
            pl.BlockSpec((_RB, 1), lambda s, i: (i, 0)),
            pl.BlockSpec((4, D), lambda s, i: (0, 0)),
            pl.BlockSpec((4, D), lambda s, i: (0, 0)),
        ],
        out_specs=[pl.BlockSpec((_RB, D), lambda s, i: (_RG * s + i, 0)),
                   pl.BlockSpec((4, D), lambda s, i: (0, 0))],
        out_shape=[jax.ShapeDtypeStruct((4 * NP, D), jnp.float32),
                   jax.ShapeDtypeStruct((4, D), jnp.float32)],
    )(parts, parts, hs, dinv, bst, ast)


def _tc_d_body(csum, wp, bp, g):
    m = jax.nn.sigmoid(csum[...] * (1.0 / N))
    g[...] = jnp.dot(m, wp[...], preferred_element_type=jnp.float32) + bp[...]


def _tc_d(csum, wp, bp):
    return pl.pallas_call(
        _tc_d_body,
        out_shape=jax.ShapeDtypeStruct((2, D), jnp.float32),
    )(csum, wp, bp)


def kernel(x, edge_index, edge_weight, W1a, b1a, W2a, b2a, a1,
           W1b, b1b, W2b, b2b, a2, Wp, bp, perm1, perm2):
    src = edge_index[0].astype(jnp.int32)
    dst = edge_index[1].astype(jnp.int32)
    w = edge_weight.astype(jnp.float32)
    p1 = perm1.astype(jnp.int32)
    p2 = perm2.astype(jnp.int32)

    degflat, xp1, xp2 = _sc_pre(dst, w, x, p1, p2)
    deg2 = degflat.reshape(NC, NP)
    d0 = deg2[0].reshape(NP, 1)
    d1 = deg2[1].reshape(NP, 1)

    pad = jnp.zeros((NP - N, D), jnp.float32)
    xpd = jnp.concatenate([x, pad], axis=0)
    xp1d = jnp.concatenate([xp1, pad], axis=0)
    xp2d = jnp.concatenate([xp2, pad], axis=0)

    hs, dinv = _tc_a(d0, d1, xpd, xp1d, xp2d, W1a, W1b)

    parts1 = _sc_mp(src, dst, w, hs)

    bst1 = jnp.stack([b1a, b1b, b1a, b1b])
    bst2 = jnp.stack([b2a, b2b, b2a, b2b])
    ast = jnp.stack([a1, a2, a1, a2])
    wst2 = jnp.stack([W2a, W2b, W2a, W2b])

    hs2 = _tc_b(parts1, hs, dinv, bst1, ast, wst2)

    parts2 = _sc_mp(src, dst, w, hs2)

    zall, csum = _tc_c(parts2, hs2, dinv, bst2, ast)

    g12 = _tc_d(csum[0:2], Wp, bp.reshape(1, D))

    z1 = zall[0:N]
    z2 = zall[NP:NP + N]
    z1n = zall[2 * NP:2 * NP + N]
    z2n = zall[3 * NP:3 * NP + N]
    return (z1, z2, g12[0:1], g12[1:2], z1n, z2n,
            jnp.arange(N, dtype=jnp.int32), x.shape[0])


# speedup vs baseline: 13.0961x; 1.1346x over previous
import functools

import jax
import jax.numpy as jnp
from jax import lax
from jax.experimental import pallas as pl
from jax.experimental.pallas import tpu as pltpu
from jax.experimental.pallas import tpu_sc as plsc

N = 10000
NP = 10240
E = 320000
D = 128
NC = 2
NS = 16
CH = 80
ECHUNKS = E // (NC * NS) // CH
GCHUNKS = N // CH
MCH = 64
MCHUNKS = 156
MTAIL = 16

_MESH = plsc.VectorSubcoreMesh(
    core_axis_name="c", subcore_axis_name="s", num_cores=NC, num_subcores=NS)


def _zero_vmem_rows(buf, nrows):
    z16 = jnp.zeros((16,), jnp.float32)

    def body(i, _):
        for j in range(D // 16):
            buf[i, pl.ds(j * 16, 16)] = z16
        return 0

    lax.fori_loop(0, nrows, body, 0)


@functools.partial(
    pl.kernel,
    out_type=[
        jax.ShapeDtypeStruct((NC * NP,), jnp.float32),
        jax.ShapeDtypeStruct((N, D), jnp.float32),
        jax.ShapeDtypeStruct((N, D), jnp.float32),
    ],
    mesh=_MESH,
    scratch_types=[
        pltpu.VMEM((CH,), jnp.int32),
        pltpu.VMEM((CH,), jnp.float32),
        pltpu.VMEM((CH, D), jnp.float32),
        pltpu.VMEM_SHARED((NP,), jnp.float32),
    ],
)
def _sc_pre(dst_hbm, w_hbm, x_hbm, p1_hbm, p2_hbm,
            deg_hbm, xp1_hbm, xp2_hbm,
            idxb, valb, rows, dacc):
    cid = lax.axis_index("c")
    sid = lax.axis_index("s")
    wid = sid * NC + cid
    zslice = NP // NS

    valb[...] = jnp.zeros((CH,), jnp.float32)

    def zbody(i, _):
        pltpu.sync_copy(valb, dacc.at[pl.ds(sid * zslice + i * CH, CH)])
        return 0

    lax.fori_loop(0, zslice // CH, zbody, 0)
    plsc.subcore_barrier()

    base_e = (cid * NS + sid) * (ECHUNKS * CH)

    def ebody(c, _):
        off = base_e + c * CH
        pltpu.sync_copy(dst_hbm.at[pl.ds(off, CH)], idxb)
        pltpu.sync_copy(w_hbm.at[pl.ds(off, CH)], valb)
        pltpu.sync_copy(valb, dacc.at[idxb], add=True)
        return 0

    lax.fori_loop(0, ECHUNKS, ebody, 0)
    plsc.subcore_barrier()

    pltpu.sync_copy(dacc.at[pl.ds(sid * zslice, zslice)],
                    deg_hbm.at[pl.ds(cid * NP + sid * zslice, zslice)])

    for i in range((GCHUNKS + NC * NS - 1) // (NC * NS)):
        c = wid + i * NC * NS

        @pl.when(c < GCHUNKS)
        def _():
            off = c * CH
            pltpu.sync_copy(p1_hbm.at[pl.ds(off, CH)], idxb)
            pltpu.sync_copy(x_hbm.at[idxb], rows)
            pltpu.sync_copy(rows, xp1_hbm.at[pl.ds(off, CH)])
            pltpu.sync_copy(p2_hbm.at[pl.ds(off, CH)], idxb)
            pltpu.sync_copy(x_hbm.at[idxb], rows)
            pltpu.sync_copy(rows, xp2_hbm.at[pl.ds(off, CH)])


@functools.partial(
    pl.kernel,
    out_type=jax.ShapeDtypeStruct((4 * NC * NP, D), jnp.float32),
    mesh=_MESH,
    scratch_types=[
        [pltpu.VMEM((MCH,), jnp.int32) for _ in range(6)],
        [pltpu.VMEM((MCH,), jnp.int32) for _ in range(6)],
        [pltpu.VMEM((MCH,), jnp.float32) for _ in range(6)],
        [pltpu.VMEM((MCH, D), jnp.float32) for _ in range(3)],
        pltpu.VMEM((MTAIL,), jnp.int32),
        pltpu.VMEM((MTAIL,), jnp.int32),
        pltpu.VMEM((MTAIL,), jnp.float32),
        pltpu.VMEM((MTAIL, D), jnp.float32),
        pltpu.VMEM_SHARED((NP, D), jnp.float32),
        [pltpu.SemaphoreType.DMA for _ in range(6)],
        [pltpu.SemaphoreType.DMA for _ in range(3)],
        pltpu.SemaphoreType.DMA,
    ],
)
def _sc_mp(src_hbm, dst_hbm, w_hbm, hs_hbm, part_hbm,
           srcb, dstb, wb, rows, tsrc, tdst, tw, trows,
           acc, isems, gsems, ssem):
    cid = lax.axis_index("c")
    sid = lax.axis_index("s")
    zslice = NP // NS
    base_e = (cid * NS + sid) * (MCHUNKS * MCH + MTAIL)

    def _scale(rows_k, wb_k, nch):
        def sbody(t, _2):
            wvec = wb_k[pl.ds(t * 16, 16)]
            for kk in range(16):
                svec = jnp.full((16,), wvec[kk], jnp.float32)
                r = t * 16 + kk
                for j in range(D // 16):
                    sl = pl.ds(j * 16, 16)
                    rows_k[r, sl] = rows_k[r, sl] * svec
            return 0

        lax.fori_loop(0, nch // 16, sbody, 0)

    def fetch_idx(c, m):
        off = base_e + c * MCH
        pltpu.async_copy(src_hbm.at[pl.ds(off, MCH)], srcb[m], isems[m])
        pltpu.async_copy(dst_hbm.at[pl.ds(off, MCH)], dstb[m], isems[m])
        pltpu.async_copy(w_hbm.at[pl.ds(off, MCH)], wb[m], isems[m])

    def wait_idx(m):
        pltpu.make_async_copy(src_hbm.at[pl.ds(0, MCH)], srcb[m],
                              isems[m]).wait()
        pltpu.make_async_copy(dst_hbm.at[pl.ds(0, MCH)], dstb[m],
                              isems[m]).wait()
        pltpu.make_async_copy(w_hbm.at[pl.ds(0, MCH)], wb[m],
                              isems[m]).wait()

    def stream_body(s, _):
        _zero_vmem_rows(rows[0], MCH)
        zds = [pltpu.async_copy(rows[0],
                                acc.at[pl.ds(sid * zslice + j * MCH, MCH)],
                                gsems[0])
               for j in range(zslice // MCH)]
        for d in zds:
            d.wait()
        plsc.subcore_barrier()

        soff = jnp.full((16,), s * NP, jnp.int32)

        def fire_gather(m, k):
            for j in range(MCH // 16):
                sl = pl.ds(j * 16, 16)
                srcb[m][sl] = srcb[m][sl] + soff
            return pltpu.async_copy(hs_hbm.at[srcb[m]], rows[k], gsems[k])

        for m in range(6):
            fetch_idx(m, m)

        def gbody(g, _g):
            c0 = g * 6
            nxt = c0 + 6
            wait_idx(0)
            gds = [fire_gather(0, 0)]
            wait_idx(1)
            gds.append(fire_gather(1, 1))
            sds = []
            for j in range(6):
                k = j % 3
                gds[j].wait()
                _scale(rows[k], wb[j], MCH)
                sds.append(pltpu.async_copy(rows[k], acc.at[dstb[j]],
                                            ssem, add=True))
                if 1 <= j <= 4:
                    sds[j - 1].wait()
                if j <= 3:
                    wait_idx(j + 2)
                    gds.append(fire_gather(j + 2, (j + 2) % 3))
                if j >= 2:
                    fetch_idx(jnp.minimum(nxt + j - 2, MCHUNKS - 1), j - 2)
            sds[4].wait()
            fetch_idx(jnp.minimum(nxt + 4, MCHUNKS - 1), 4)
            sds[5].wait()
            fetch_idx(jnp.minimum(nxt + 5, MCHUNKS - 1), 5)
            return 0

        lax.fori_loop(0, MCHUNKS // 6, gbody, 0)

        for m in range(6):
            wait_idx(m)

        toff = base_e + MCHUNKS * MCH
        pltpu.sync_copy(src_hbm.at[pl.ds(toff, MTAIL)], tsrc)
        pltpu.sync_copy(dst_hbm.at[pl.ds(toff, MTAIL)], tdst)
        pltpu.sync_copy(w_hbm.at[pl.ds(toff, MTAIL)], tw)
        tsrc[pl.ds(0, 16)] = tsrc[pl.ds(0, 16)] + soff
        pltpu.sync_copy(hs_hbm.at[tsrc], trows)
        _scale(trows, tw, MTAIL)
        pltpu.sync_copy(trows, acc.at[tdst], add=True)
        plsc.subcore_barrier()

        dds = []
        for j in range(zslice // MCH):
            roff = sid * zslice + j * MCH
            dds.append(pltpu.async_copy(
                acc.at[pl.ds(roff, MCH)],
                part_hbm.at[pl.ds(s * (NC * NP) + cid * NP + roff, MCH)],
                gsems[1]))
        for d in dds:
            d.wait()
        plsc.subcore_barrier()
        return 0

    lax.fori_loop(0, 4, stream_body, 0)


def _prelu(z, a):
    return jnp.maximum(z, 0.0) + a * jnp.minimum(z, 0.0)


def _tc_a_body(d0, d1, x, xp1, xp2, w1a, w1b, hs, dinv):
    dv = lax.rsqrt(d0[...] + d1[...] + 1.0)
    dinv[...] = dv
    hs[0:NP, :] = jnp.dot(x[...], w1a[...],
                          preferred_element_type=jnp.float32) * dv
    hs[NP:2 * NP, :] = jnp.dot(x[...], w1b[...],
                               preferred_element_type=jnp.float32) * dv
    hs[2 * NP:3 * NP, :] = jnp.dot(xp1[...], w1a[...],
                                   preferred_element_type=jnp.float32) * dv
    hs[3 * NP:4 * NP, :] = jnp.dot(xp2[...], w1b[...],
                                   preferred_element_type=jnp.float32) * dv


def _tc_a(d0, d1, x, xp1, xp2, w1a, w1b):
    return pl.pallas_call(
        _tc_a_body,
        out_shape=[jax.ShapeDtypeStruct((4 * NP, D), jnp.float32),
                   jax.ShapeDtypeStruct((NP, 1), jnp.float32)],
    )(d0, d1, x, xp1, xp2, w1a, w1b)


_RB = 2048
_RG = NP // _RB


def _tc_b_body(pa, pb, h, dinv, bst, ast, wst, o):
    s = pl.program_id(0)
    dv = dinv[...]
    agg = dv * (pa[...] + pb[...] + h[...]) + bst[pl.ds(s, 1), :]
    z = _prelu(agg, ast[pl.ds(s, 1), :])
    wmat = wst[pl.ds(s, 1), :, :][0]
    o[...] = jnp.dot(z, wmat, preferred_element_type=jnp.float32) * dv


def _tc_b(parts, hs, dinv, bst, ast, wst):
    return pl.pallas_call(
        _tc_b_body,
        grid=(4, _RG),
        in_specs=[
            pl.BlockSpec((_RB, D), lambda s, i: (2 * _RG * s + i, 0)),
            pl.BlockSpec((_RB, D), lambda s, i: (2 * _RG * s + _RG + i, 0)),
            pl.BlockSpec((_RB, D), lambda s, i: (_RG * s + i, 0)),
            pl.BlockSpec((_RB, 1), lambda s, i: (i, 0)),
            pl.BlockSpec((4, D), lambda s, i: (0, 0)),
            pl.BlockSpec((4, D), lambda s, i: (0, 0)),
            pl.BlockSpec((4, D, D), lambda s, i: (0, 0, 0)),
        ],
        out_specs=pl.BlockSpec((_RB, D), lambda s, i: (_RG * s + i, 0)),
        out_shape=jax.ShapeDtypeStruct((4 * NP, D), jnp.float32),
    )(parts, parts, hs, dinv, bst, ast, wst)


def _tc_c_body(pa, pb, h, dinv, bst, ast, o, csum):
    s = pl.program_id(0)
    i = pl.program_id(1)
    dv = dinv[...]
    agg = dv * (pa[...] + pb[...] + h[...]) + bst[pl.ds(s, 1), :]
    z = _prelu(agg, ast[pl.ds(s, 1), :])
    o[...] = z

    @pl.when(jnp.logical_and(s == 0, i == 0))
    def _():
        csum[...] = jnp.zeros_like(csum)

    rid = lax.broadcasted_iota(jnp.int32, (_RB, 1), 0) + i * _RB
    zm = jnp.where(rid < N, z, 0.0)
    csum[pl.ds(s, 1), :] = (csum[pl.ds(s, 1), :]
                            + jnp.sum(zm, axis=0, keepdims=True))


def _tc_c(parts, hs, dinv, bst, ast):
    return pl.pallas_call(
        _tc_c_body,
        grid=(4, _RG),
        in_specs=[
            pl.BlockSpec((_RB, D), lambda s, i: (2 * _RG * s + i, 0)),
            pl.BlockSpec((_RB, D), lambda s, i: (2 * _RG * s + _RG + i, 0)),
            pl.BlockSpec((_RB, D), lambda s, i: (_RG * s + i, 0)),
            pl.BlockSpec((_RB, 1), lambda s, i: (i, 0)),
            pl.BlockSpec((4, D), lambda s, i: (0, 0)),
            pl.BlockSpec((4, D), lambda s, i: (0, 0)),
        ],
        out_specs=[pl.BlockSpec((_RB, D), lambda s, i: (_RG * s + i, 0)),
                   pl.BlockSpec((4, D), lambda s, i: (0, 0))],
        out_shape=[jax.ShapeDtypeStruct((4 * NP, D), jnp.float32),
                   jax.ShapeDtypeStruct((4, D), jnp.float32)],
    )(parts, parts, hs, dinv, bst, ast)


def _tc_d_body(csum, wp, bp, g):
    m = jax.nn.sigmoid(csum[...] * (1.0 / N))
    g[...] = jnp.dot(m, wp[...], preferred_element_type=jnp.float32) + bp[...]


def _tc_d(csum, wp, bp):
    return pl.pallas_call(
        _tc_d_body,
        out_shape=jax.ShapeDtypeStruct((2, D), jnp.float32),
    )(csum, wp, bp)


def kernel(x, edge_index, edge_weight, W1a, b1a, W2a, b2a, a1,
           W1b, b1b, W2b, b2b, a2, Wp, bp, perm1, perm2):
    src = edge_index[0].astype(jnp.int32)
    dst = edge_index[1].astype(jnp.int32)
    w = edge_weight.astype(jnp.float32)
    p1 = perm1.astype(jnp.int32)
    p2 = perm2.astype(jnp.int32)

    degflat, xp1, xp2 = _sc_pre(dst, w, x, p1, p2)
    deg2 = degflat.reshape(NC, NP)
    d0 = deg2[0].reshape(NP, 1)
    d1 = deg2[1].reshape(NP, 1)

    pad = jnp.zeros((NP - N, D), jnp.float32)
    xpd = jnp.concatenate([x, pad], axis=0)
    xp1d = jnp.concatenate([xp1, pad], axis=0)
    xp2d = jnp.concatenate([xp2, pad], axis=0)

    hs, dinv = _tc_a(d0, d1, xpd, xp1d, xp2d, W1a, W1b)

    parts1 = _sc_mp(src, dst, w, hs)

    bst1 = jnp.stack([b1a, b1b, b1a, b1b])
    bst2 = jnp.stack([b2a, b2b, b2a, b2b])
    ast = jnp.stack([a1, a2, a1, a2])
    wst2 = jnp.stack([W2a, W2b, W2a, W2b])

    hs2 = _tc_b(parts1, hs, dinv, bst1, ast, wst2)

    parts2 = _sc_mp(src, dst, w, hs2)

    zall, csum = _tc_c(parts2, hs2, dinv, bst2, ast)

    g12 = _tc_d(csum[0:2], Wp, bp.reshape(1, D))

    z1 = zall[0:N]
    z2 = zall[NP:NP + N]
    z1n = zall[2 * NP:2 * NP + N]
    z2n = zall[3 * NP:3 * NP + N]
    return (z1, z2, g12[0:1], g12[1:2], z1n, z2n,
            jnp.arange(N, dtype=jnp.int32), x.shape[0])


# speedup vs baseline: 13.9604x; 1.0660x over previous
import functools

import jax
import jax.numpy as jnp
from jax import lax
from jax.experimental import pallas as pl
from jax.experimental.pallas import tpu as pltpu
from jax.experimental.pallas import tpu_sc as plsc

N = 10000
NP = 10240
E = 320000
D = 128
NC = 2
NS = 16
CH = 80
ECHUNKS = E // (NC * NS) // CH
GCHUNKS = N // CH
MCH = 64
MCHUNKS = 156
MTAIL = 16

_MESH = plsc.VectorSubcoreMesh(
    core_axis_name="c", subcore_axis_name="s", num_cores=NC, num_subcores=NS)


def _zero_vmem_rows(buf, nrows):
    z16 = jnp.zeros((16,), jnp.float32)

    def body(i, _):
        for j in range(D // 16):
            buf[i, pl.ds(j * 16, 16)] = z16
        return 0

    lax.fori_loop(0, nrows, body, 0)


DCH = 128
DCHUNKS = 78


@functools.partial(
    pl.kernel,
    out_type=[
        jax.ShapeDtypeStruct((NC * NP,), jnp.float32),
        jax.ShapeDtypeStruct((N, D), jnp.float32),
        jax.ShapeDtypeStruct((N, D), jnp.float32),
    ],
    mesh=_MESH,
    scratch_types=[
        pltpu.VMEM((CH,), jnp.int32),
        pltpu.VMEM((CH,), jnp.float32),
        pltpu.VMEM((CH, D), jnp.float32),
        [pltpu.VMEM((DCH,), jnp.int32) for _ in range(2)],
        [pltpu.VMEM((DCH,), jnp.float32) for _ in range(2)],
        pltpu.VMEM((16,), jnp.int32),
        pltpu.VMEM((16,), jnp.float32),
        pltpu.VMEM_SHARED((NP,), jnp.float32),
        [pltpu.SemaphoreType.DMA for _ in range(2)],
        [pltpu.SemaphoreType.DMA for _ in range(2)],
    ],
)
def _sc_pre(dst_hbm, w_hbm, x_hbm, p1_hbm, p2_hbm,
            deg_hbm, xp1_hbm, xp2_hbm,
            idxb, valb, rows, db, vb, tdb, tvb, dacc, isems, ssems):
    cid = lax.axis_index("c")
    sid = lax.axis_index("s")
    wid = sid * NC + cid
    zslice = NP // NS

    valb[...] = jnp.zeros((CH,), jnp.float32)

    def zbody(i, _):
        pltpu.sync_copy(valb, dacc.at[pl.ds(sid * zslice + i * CH, CH)])
        return 0

    lax.fori_loop(0, zslice // CH, zbody, 0)
    plsc.subcore_barrier()

    base_e = (cid * NS + sid) * (DCHUNKS * DCH + 16)

    def fetch(c, k):
        off = base_e + c * DCH
        pltpu.async_copy(dst_hbm.at[pl.ds(off, DCH)], db[k], isems[k])
        pltpu.async_copy(w_hbm.at[pl.ds(off, DCH)], vb[k], isems[k])

    def wait_fetch(k):
        pltpu.make_async_copy(dst_hbm.at[pl.ds(0, DCH)], db[k],
                              isems[k]).wait()
        pltpu.make_async_copy(w_hbm.at[pl.ds(0, DCH)], vb[k],
                              isems[k]).wait()

    fetch(0, 0)
    fetch(1, 1)

    def ebody(g, _):
        c0 = g * 2
        wait_fetch(0)
        s0 = pltpu.async_copy(vb[0], dacc.at[db[0]], ssems[0], add=True)
        wait_fetch(1)
        s1 = pltpu.async_copy(vb[1], dacc.at[db[1]], ssems[1], add=True)
        s0.wait()
        fetch(jnp.minimum(c0 + 2, DCHUNKS - 1), 0)
        s1.wait()
        fetch(jnp.minimum(c0 + 3, DCHUNKS - 1), 1)
        return 0

    lax.fori_loop(0, DCHUNKS // 2, ebody, 0)
    wait_fetch(0)
    wait_fetch(1)

    toff = base_e + DCHUNKS * DCH
    pltpu.sync_copy(dst_hbm.at[pl.ds(toff, 16)], tdb)
    pltpu.sync_copy(w_hbm.at[pl.ds(toff, 16)], tvb)
    pltpu.sync_copy(tvb, dacc.at[tdb], add=True)
    plsc.subcore_barrier()

    pltpu.sync_copy(dacc.at[pl.ds(sid * zslice, zslice)],
                    deg_hbm.at[pl.ds(cid * NP + sid * zslice, zslice)])

    for i in range((GCHUNKS + NC * NS - 1) // (NC * NS)):
        c = wid + i * NC * NS

        @pl.when(c < GCHUNKS)
        def _():
            off = c * CH
            pltpu.sync_copy(p1_hbm.at[pl.ds(off, CH)], idxb)
            pltpu.sync_copy(x_hbm.at[idxb], rows)
            pltpu.sync_copy(rows, xp1_hbm.at[pl.ds(off, CH)])
            pltpu.sync_copy(p2_hbm.at[pl.ds(off, CH)], idxb)
            pltpu.sync_copy(x_hbm.at[idxb], rows)
            pltpu.sync_copy(rows, xp2_hbm.at[pl.ds(off, CH)])


@functools.partial(
    pl.kernel,
    out_type=jax.ShapeDtypeStruct((4 * NC * NP, D), jnp.float32),
    mesh=_MESH,
    scratch_types=[
        [pltpu.VMEM((MCH,), jnp.int32) for _ in range(6)],
        [pltpu.VMEM((MCH,), jnp.int32) for _ in range(6)],
        [pltpu.VMEM((MCH,), jnp.float32) for _ in range(6)],
        [pltpu.VMEM((MCH, D), jnp.float32) for _ in range(3)],
        pltpu.VMEM((MTAIL,), jnp.int32),
        pltpu.VMEM((MTAIL,), jnp.int32),
        pltpu.VMEM((MTAIL,), jnp.float32),
        pltpu.VMEM((MTAIL, D), jnp.float32),
        pltpu.VMEM_SHARED((NP, D), jnp.float32),
        [pltpu.SemaphoreType.DMA for _ in range(6)],
        [pltpu.SemaphoreType.DMA for _ in range(3)],
        pltpu.SemaphoreType.DMA,
    ],
)
def _sc_mp(src_hbm, dst_hbm, w_hbm, hs_hbm, part_hbm,
           srcb, dstb, wb, rows, tsrc, tdst, tw, trows,
           acc, isems, gsems, ssem):
    cid = lax.axis_index("c")
    sid = lax.axis_index("s")
    zslice = NP // NS
    base_e = (cid * NS + sid) * (MCHUNKS * MCH + MTAIL)

    def _scale(rows_k, wb_k, nch):
        def sbody(t, _2):
            wvec = wb_k[pl.ds(t * 16, 16)]
            for kk in range(16):
                svec = jnp.full((16,), wvec[kk], jnp.float32)
                r = t * 16 + kk
                for j in range(D // 16):
                    sl = pl.ds(j * 16, 16)
                    rows_k[r, sl] = rows_k[r, sl] * svec
            return 0

        lax.fori_loop(0, nch // 16, sbody, 0)

    def fetch_idx(c, m):
        off = base_e + c * MCH
        pltpu.async_copy(src_hbm.at[pl.ds(off, MCH)], srcb[m], isems[m])
        pltpu.async_copy(dst_hbm.at[pl.ds(off, MCH)], dstb[m], isems[m])
        pltpu.async_copy(w_hbm.at[pl.ds(off, MCH)], wb[m], isems[m])

    def wait_idx(m):
        pltpu.make_async_copy(src_hbm.at[pl.ds(0, MCH)], srcb[m],
                              isems[m]).wait()
        pltpu.make_async_copy(dst_hbm.at[pl.ds(0, MCH)], dstb[m],
                              isems[m]).wait()
        pltpu.make_async_copy(w_hbm.at[pl.ds(0, MCH)], wb[m],
                              isems[m]).wait()

    def stream_body(s, _):
        _zero_vmem_rows(rows[0], MCH)
        zds = [pltpu.async_copy(rows[0],
                                acc.at[pl.ds(sid * zslice + j * MCH, MCH)],
                                gsems[0])
               for j in range(zslice // MCH)]
        for d in zds:
            d.wait()
        plsc.subcore_barrier()

        soff = jnp.full((16,), s * NP, jnp.int32)

        def fire_gather(m, k):
            for j in range(MCH // 16):
                sl = pl.ds(j * 16, 16)
                srcb[m][sl] = srcb[m][sl] + soff
            return pltpu.async_copy(hs_hbm.at[srcb[m]], rows[k], gsems[k])

        for m in range(6):
            fetch_idx(m, m)

        def gbody(g, _g):
            c0 = g * 6
            nxt = c0 + 6
            wait_idx(0)
            gds = [fire_gather(0, 0)]
            wait_idx(1)
            gds.append(fire_gather(1, 1))
            sds = []
            for j in range(6):
                k = j % 3
                gds[j].wait()
                _scale(rows[k], wb[j], MCH)
                sds.append(pltpu.async_copy(rows[k], acc.at[dstb[j]],
                                            ssem, add=True))
                if 1 <= j <= 4:
                    sds[j - 1].wait()
                if j <= 3:
                    wait_idx(j + 2)
                    gds.append(fire_gather(j + 2, (j + 2) % 3))
                if j >= 2:
                    fetch_idx(jnp.minimum(nxt + j - 2, MCHUNKS - 1), j - 2)
            sds[4].wait()
            fetch_idx(jnp.minimum(nxt + 4, MCHUNKS - 1), 4)
            sds[5].wait()
            fetch_idx(jnp.minimum(nxt + 5, MCHUNKS - 1), 5)
            return 0

        lax.fori_loop(0, MCHUNKS // 6, gbody, 0)

        for m in range(6):
            wait_idx(m)

        toff = base_e + MCHUNKS * MCH
        pltpu.sync_copy(src_hbm.at[pl.ds(toff, MTAIL)], tsrc)
        pltpu.sync_copy(dst_hbm.at[pl.ds(toff, MTAIL)], tdst)
        pltpu.sync_copy(w_hbm.at[pl.ds(toff, MTAIL)], tw)
        tsrc[pl.ds(0, 16)] = tsrc[pl.ds(0, 16)] + soff
        pltpu.sync_copy(hs_hbm.at[tsrc], trows)
        _scale(trows, tw, MTAIL)
        pltpu.sync_copy(trows, acc.at[tdst], add=True)
        plsc.subcore_barrier()

        dds = []
        for j in range(zslice // MCH):
            roff = sid * zslice + j * MCH
            dds.append(pltpu.async_copy(
                acc.at[pl.ds(roff, MCH)],
                part_hbm.at[pl.ds(s * (NC * NP) + cid * NP + roff, MCH)],
                gsems[1]))
        for d in dds:
            d.wait()
        plsc.subcore_barrier()
        return 0

    lax.fori_loop(0, 4, stream_body, 0)


def _prelu(z, a):
    return jnp.maximum(z, 0.0) + a * jnp.minimum(z, 0.0)


def _tc_a_body(d0, d1, x, xp1, xp2, w1a, w1b, hs, dinv):
    dv = lax.rsqrt(d0[...] + d1[...] + 1.0)
    dinv[...] = dv
    hs[0:NP, :] = jnp.dot(x[...], w1a[...],
                          preferred_element_type=jnp.float32) * dv
    hs[NP:2 * NP, :] = jnp.dot(x[...], w1b[...],
                               preferred_element_type=jnp.float32) * dv
    hs[2 * NP:3 * NP, :] = jnp.dot(xp1[...], w1a[...],
                                   preferred_element_type=jnp.float32) * dv
    hs[3 * NP:4 * NP, :] = jnp.dot(xp2[...], w1b[...],
                                   preferred_element_type=jnp.float32) * dv


def _tc_a(d0, d1, x, xp1, xp2, w1a, w1b):
    return pl.pallas_call(
        _tc_a_body,
        out_shape=[jax.ShapeDtypeStruct((4 * NP, D), jnp.float32),
                   jax.ShapeDtypeStruct((NP, 1), jnp.float32)],
    )(d0, d1, x, xp1, xp2, w1a, w1b)


_RB = 2048
_RG = NP // _RB


def _tc_b_body(pa, pb, h, dinv, bst, ast, wst, o):
    s = pl.program_id(0)
    dv = dinv[...]
    agg = dv * (pa[...] + pb[...] + h[...]) + bst[pl.ds(s, 1), :]
    z = _prelu(agg, ast[pl.ds(s, 1), :])
    wmat = wst[pl.ds(s, 1), :, :][0]
    o[...] = jnp.dot(z, wmat, preferred_element_type=jnp.float32) * dv


def _tc_b(parts, hs, dinv, bst, ast, wst):
    return pl.pallas_call(
        _tc_b_body,
        grid=(4, _RG),
        in_specs=[
            pl.BlockSpec((_RB, D), lambda s, i: (2 * _RG * s + i, 0)),
            pl.BlockSpec((_RB, D), lambda s, i: (2 * _RG * s + _RG + i, 0)),
            pl.BlockSpec((_RB, D), lambda s, i: (_RG * s + i, 0)),
            pl.BlockSpec((_RB, 1), lambda s, i: (i, 0)),
            pl.BlockSpec((4, D), lambda s, i: (0, 0)),
            pl.BlockSpec((4, D), lambda s, i: (0, 0)),
            pl.BlockSpec((4, D, D), lambda s, i: (0, 0, 0)),
        ],
        out_specs=pl.BlockSpec((_RB, D), lambda s, i: (_RG * s + i, 0)),
        out_shape=jax.ShapeDtypeStruct((4 * NP, D), jnp.float32),
    )(parts, parts, hs, dinv, bst, ast, wst)


def _tc_c_body(pa, pb, h, dinv, bst, ast, o, csum):
    s = pl.program_id(0)
    i = pl.program_id(1)
    dv = dinv[...]
    agg = dv * (pa[...] + pb[...] + h[...]) + bst[pl.ds(s, 1), :]
    z = _prelu(agg, ast[pl.ds(s, 1), :])
    o[...] = z

    @pl.when(jnp.logical_and(s == 0, i == 0))
    def _():
        csum[...] = jnp.zeros_like(csum)

    rid = lax.broadcasted_iota(jnp.int32, (_RB, 1), 0) + i * _RB
    zm = jnp.where(rid < N, z, 0.0)
    csum[pl.ds(s, 1), :] = (csum[pl.ds(s, 1), :]
                            + jnp.sum(zm, axis=0, keepdims=True))


def _tc_c(parts, hs, dinv, bst, ast):
    return pl.pallas_call(
        _tc_c_body,
        grid=(4, _RG),
        in_specs=[
            pl.BlockSpec((_RB, D), lambda s, i: (2 * _RG * s + i, 0)),
            pl.BlockSpec((_RB, D), lambda s, i: (2 * _RG * s + _RG + i, 0)),
            pl.BlockSpec((_RB, D), lambda s, i: (_RG * s + i, 0)),
            pl.BlockSpec((_RB, 1), lambda s, i: (i, 0)),
            pl.BlockSpec((4, D), lambda s, i: (0, 0)),
            pl.BlockSpec((4, D), lambda s, i: (0, 0)),
        ],
        out_specs=[pl.BlockSpec((_RB, D), lambda s, i: (_RG * s + i, 0)),
                   pl.BlockSpec((4, D), lambda s, i: (0, 0))],
        out_shape=[jax.ShapeDtypeStruct((4 * NP, D), jnp.float32),
                   jax.ShapeDtypeStruct((4, D), jnp.float32)],
    )(parts, parts, hs, dinv, bst, ast)


def _tc_d_body(csum, wp, bp, g):
    m = jax.nn.sigmoid(csum[...] * (1.0 / N))
    g[...] = jnp.dot(m, wp[...], preferred_element_type=jnp.float32) + bp[...]


def _tc_d(csum, wp, bp):
    return pl.pallas_call(
        _tc_d_body,
        out_shape=jax.ShapeDtypeStruct((2, D), jnp.float32),
    )(csum, wp, bp)


def kernel(x, edge_index, edge_weight, W1a, b1a, W2a, b2a, a1,
           W1b, b1b, W2b, b2b, a2, Wp, bp, perm1, perm2):
    src = edge_index[0].astype(jnp.int32)
    dst = edge_index[1].astype(jnp.int32)
    w = edge_weight.astype(jnp.float32)
    p1 = perm1.astype(jnp.int32)
    p2 = perm2.astype(jnp.int32)

    degflat, xp1, xp2 = _sc_pre(dst, w, x, p1, p2)
    deg2 = degflat.reshape(NC, NP)
    d0 = deg2[0].reshape(NP, 1)
    d1 = deg2[1].reshape(NP, 1)

    pad = jnp.zeros((NP - N, D), jnp.float32)
    xpd = jnp.concatenate([x, pad], axis=0)
    xp1d = jnp.concatenate([xp1, pad], axis=0)
    xp2d = jnp.concatenate([xp2, pad], axis=0)

    hs, dinv = _tc_a(d0, d1, xpd, xp1d, xp2d, W1a, W1b)

    parts1 = _sc_mp(src, dst, w, hs)

    bst1 = jnp.stack([b1a, b1b, b1a, b1b])
    bst2 = jnp.stack([b2a, b2b, b2a, b2b])
    ast = jnp.stack([a1, a2, a1, a2])
    wst2 = jnp.stack([W2a, W2b, W2a, W2b])

    hs2 = _tc_b(parts1, hs, dinv, bst1, ast, wst2)

    parts2 = _sc_mp(src, dst, w, hs2)

    zall, csum = _tc_c(parts2, hs2, dinv, bst2, ast)

    g12 = _tc_d(csum[0:2], Wp, bp.reshape(1, D))

    z1 = zall[0:N]
    z2 = zall[NP:NP + N]
    z1n = zall[2 * NP:2 * NP + N]
    z2n = zall[3 * NP:3 * NP + N]
    return (z1, z2, g12[0:1], g12[1:2], z1n, z2n,
            jnp.arange(N, dtype=jnp.int32), x.shape[0])


# speedup vs baseline: 13.9660x; 1.0004x over previous
import functools

import jax
import jax.numpy as jnp
from jax import lax
from jax.experimental import pallas as pl
from jax.experimental.pallas import tpu as pltpu
from jax.experimental.pallas import tpu_sc as plsc

N = 10000
NP = 10240
E = 320000
D = 128
NC = 2
NS = 16
CH = 80
ECHUNKS = E // (NC * NS) // CH
GCHUNKS = N // CH
MCH = 64
MCHUNKS = 156
MTAIL = 16

_MESH = plsc.VectorSubcoreMesh(
    core_axis_name="c", subcore_axis_name="s", num_cores=NC, num_subcores=NS)


def _zero_vmem_rows(buf, nrows):
    z16 = jnp.zeros((16,), jnp.float32)

    def body(i, _):
        for j in range(D // 16):
            buf[i, pl.ds(j * 16, 16)] = z16
        return 0

    lax.fori_loop(0, nrows, body, 0)


DCH = 128
DCHUNKS = 78


@functools.partial(
    pl.kernel,
    out_type=[
        jax.ShapeDtypeStruct((NC * NP,), jnp.float32),
        jax.ShapeDtypeStruct((N, D), jnp.float32),
        jax.ShapeDtypeStruct((N, D), jnp.float32),
    ],
    mesh=_MESH,
    scratch_types=[
        pltpu.VMEM((CH,), jnp.int32),
        pltpu.VMEM((CH,), jnp.float32),
        pltpu.VMEM((CH, D), jnp.float32),
        [pltpu.VMEM((DCH,), jnp.int32) for _ in range(2)],
        [pltpu.VMEM((DCH,), jnp.float32) for _ in range(2)],
        pltpu.VMEM((16,), jnp.int32),
        pltpu.VMEM((16,), jnp.float32),
        pltpu.VMEM_SHARED((NP,), jnp.float32),
        [pltpu.SemaphoreType.DMA for _ in range(2)],
        [pltpu.SemaphoreType.DMA for _ in range(2)],
    ],
)
def _sc_pre(dst_hbm, w_hbm, x_hbm, p1_hbm, p2_hbm,
            deg_hbm, xp1_hbm, xp2_hbm,
            idxb, valb, rows, db, vb, tdb, tvb, dacc, isems, ssems):
    cid = lax.axis_index("c")
    sid = lax.axis_index("s")
    wid = sid * NC + cid
    zslice = NP // NS

    valb[...] = jnp.zeros((CH,), jnp.float32)

    def zbody(i, _):
        pltpu.sync_copy(valb, dacc.at[pl.ds(sid * zslice + i * CH, CH)])
        return 0

    lax.fori_loop(0, zslice // CH, zbody, 0)
    plsc.subcore_barrier()

    base_e = (cid * NS + sid) * (DCHUNKS * DCH + 16)

    def fetch(c, k):
        off = base_e + c * DCH
        pltpu.async_copy(dst_hbm.at[pl.ds(off, DCH)], db[k], isems[k])
        pltpu.async_copy(w_hbm.at[pl.ds(off, DCH)], vb[k], isems[k])

    def wait_fetch(k):
        pltpu.make_async_copy(dst_hbm.at[pl.ds(0, DCH)], db[k],
                              isems[k]).wait()
        pltpu.make_async_copy(w_hbm.at[pl.ds(0, DCH)], vb[k],
                              isems[k]).wait()

    fetch(0, 0)
    fetch(1, 1)

    def ebody(g, _):
        c0 = g * 2
        wait_fetch(0)
        s0 = pltpu.async_copy(vb[0], dacc.at[db[0]], ssems[0], add=True)
        wait_fetch(1)
        s1 = pltpu.async_copy(vb[1], dacc.at[db[1]], ssems[1], add=True)
        s0.wait()
        fetch(jnp.minimum(c0 + 2, DCHUNKS - 1), 0)
        s1.wait()
        fetch(jnp.minimum(c0 + 3, DCHUNKS - 1), 1)
        return 0

    lax.fori_loop(0, DCHUNKS // 2, ebody, 0)
    wait_fetch(0)
    wait_fetch(1)

    toff = base_e + DCHUNKS * DCH
    pltpu.sync_copy(dst_hbm.at[pl.ds(toff, 16)], tdb)
    pltpu.sync_copy(w_hbm.at[pl.ds(toff, 16)], tvb)
    pltpu.sync_copy(tvb, dacc.at[tdb], add=True)
    plsc.subcore_barrier()

    pltpu.sync_copy(dacc.at[pl.ds(sid * zslice, zslice)],
                    deg_hbm.at[pl.ds(cid * NP + sid * zslice, zslice)])

    for i in range((GCHUNKS + NC * NS - 1) // (NC * NS)):
        c = wid + i * NC * NS

        @pl.when(c < GCHUNKS)
        def _():
            off = c * CH
            pltpu.sync_copy(p1_hbm.at[pl.ds(off, CH)], idxb)
            pltpu.sync_copy(x_hbm.at[idxb], rows)
            pltpu.sync_copy(rows, xp1_hbm.at[pl.ds(off, CH)])
            pltpu.sync_copy(p2_hbm.at[pl.ds(off, CH)], idxb)
            pltpu.sync_copy(x_hbm.at[idxb], rows)
            pltpu.sync_copy(rows, xp2_hbm.at[pl.ds(off, CH)])


@functools.partial(
    pl.kernel,
    out_type=jax.ShapeDtypeStruct((4 * NC * NP, D), jnp.float32),
    mesh=_MESH,
    scratch_types=[
        [pltpu.VMEM((MCH,), jnp.int32) for _ in range(6)],
        [pltpu.VMEM((MCH,), jnp.int32) for _ in range(6)],
        [pltpu.VMEM((MCH,), jnp.float32) for _ in range(6)],
        [pltpu.VMEM((MCH, D), jnp.float32) for _ in range(3)],
        pltpu.VMEM((MTAIL,), jnp.int32),
        pltpu.VMEM((MTAIL,), jnp.int32),
        pltpu.VMEM((MTAIL,), jnp.float32),
        pltpu.VMEM((MTAIL, D), jnp.float32),
        pltpu.VMEM_SHARED((NP, D), jnp.float32),
        [pltpu.SemaphoreType.DMA for _ in range(6)],
        [pltpu.SemaphoreType.DMA for _ in range(3)],
        pltpu.SemaphoreType.DMA,
    ],
)
def _sc_mp(src_hbm, dst_hbm, w_hbm, hs_hbm, part_hbm,
           srcb, dstb, wb, rows, tsrc, tdst, tw, trows,
           acc, isems, gsems, ssem):
    cid = lax.axis_index("c")
    sid = lax.axis_index("s")
    zslice = NP // NS
    base_e = (cid * NS + sid) * (MCHUNKS * MCH + MTAIL)

    def _scale(rows_k, wb_k, nch):
        def sbody(t, _2):
            wvec = wb_k[pl.ds(t * 16, 16)]
            svecs = [jnp.full((16,), wvec[kk], jnp.float32)
                     for kk in range(16)]
            for kk in range(16):
                r = t * 16 + kk
                for j in range(D // 16):
                    sl = pl.ds(j * 16, 16)
                    rows_k[r, sl] = rows_k[r, sl] * svecs[kk]
            return 0

        lax.fori_loop(0, nch // 16, sbody, 0)

    def fetch_idx(c, m):
        off = base_e + c * MCH
        pltpu.async_copy(src_hbm.at[pl.ds(off, MCH)], srcb[m], isems[m])
        pltpu.async_copy(dst_hbm.at[pl.ds(off, MCH)], dstb[m], isems[m])
        pltpu.async_copy(w_hbm.at[pl.ds(off, MCH)], wb[m], isems[m])

    def wait_idx(m):
        pltpu.make_async_copy(src_hbm.at[pl.ds(0, MCH)], srcb[m],
                              isems[m]).wait()
        pltpu.make_async_copy(dst_hbm.at[pl.ds(0, MCH)], dstb[m],
                              isems[m]).wait()
        pltpu.make_async_copy(w_hbm.at[pl.ds(0, MCH)], wb[m],
                              isems[m]).wait()

    def stream_body(s, _):
        _zero_vmem_rows(rows[0], MCH)
        zds = [pltpu.async_copy(rows[0],
                                acc.at[pl.ds(sid * zslice + j * MCH, MCH)],
                                gsems[0])
               for j in range(zslice // MCH)]
        for d in zds:
            d.wait()
        plsc.subcore_barrier()

        soff = jnp.full((16,), s * NP, jnp.int32)

        def fire_gather(m, k):
            for j in range(MCH // 16):
                sl = pl.ds(j * 16, 16)
                srcb[m][sl] = srcb[m][sl] + soff
            return pltpu.async_copy(hs_hbm.at[srcb[m]], rows[k], gsems[k])

        for m in range(6):
            fetch_idx(m, m)

        def gbody(g, _g):
            c0 = g * 6
            nxt = c0 + 6
            wait_idx(0)
            gds = [fire_gather(0, 0)]
            wait_idx(1)
            gds.append(fire_gather(1, 1))
            sds = []
            for j in range(6):
                k = j % 3
                gds[j].wait()
                _scale(rows[k], wb[j], MCH)
                sds.append(pltpu.async_copy(rows[k], acc.at[dstb[j]],
                                            ssem, add=True))
                if 1 <= j <= 4:
                    sds[j - 1].wait()
                if j <= 3:
                    wait_idx(j + 2)
                    gds.append(fire_gather(j + 2, (j + 2) % 3))
                if j >= 2:
                    fetch_idx(jnp.minimum(nxt + j - 2, MCHUNKS - 1), j - 2)
            sds[4].wait()
            fetch_idx(jnp.minimum(nxt + 4, MCHUNKS - 1), 4)
            sds[5].wait()
            fetch_idx(jnp.minimum(nxt + 5, MCHUNKS - 1), 5)
            return 0

        lax.fori_loop(0, MCHUNKS // 6, gbody, 0)

        for m in range(6):
            wait_idx(m)

        toff = base_e + MCHUNKS * MCH
        pltpu.sync_copy(src_hbm.at[pl.ds(toff, MTAIL)], tsrc)
        pltpu.sync_copy(dst_hbm.at[pl.ds(toff, MTAIL)], tdst)
        pltpu.sync_copy(w_hbm.at[pl.ds(toff, MTAIL)], tw)
        tsrc[pl.ds(0, 16)] = tsrc[pl.ds(0, 16)] + soff
        pltpu.sync_copy(hs_hbm.at[tsrc], trows)
        _scale(trows, tw, MTAIL)
        pltpu.sync_copy(trows, acc.at[tdst], add=True)
        plsc.subcore_barrier()

        dds = []
        for j in range(zslice // MCH):
            roff = sid * zslice + j * MCH
            dds.append(pltpu.async_copy(
                acc.at[pl.ds(roff, MCH)],
                part_hbm.at[pl.ds(s * (NC * NP) + cid * NP + roff, MCH)],
                gsems[1]))
        for d in dds:
            d.wait()
        plsc.subcore_barrier()
        return 0

    lax.fori_loop(0, 4, stream_body, 0)


def _prelu(z, a):
    return jnp.maximum(z, 0.0) + a * jnp.minimum(z, 0.0)


def _tc_a_body(d0, d1, x, xp1, xp2, w1a, w1b, hs, dinv):
    dv = lax.rsqrt(d0[...] + d1[...] + 1.0)
    dinv[...] = dv
    hs[0:NP, :] = jnp.dot(x[...], w1a[...],
                          preferred_element_type=jnp.float32) * dv
    hs[NP:2 * NP, :] = jnp.dot(x[...], w1b[...],
                               preferred_element_type=jnp.float32) * dv
    hs[2 * NP:3 * NP, :] = jnp.dot(xp1[...], w1a[...],
                                   preferred_element_type=jnp.float32) * dv
    hs[3 * NP:4 * NP, :] = jnp.dot(xp2[...], w1b[...],
                                   preferred_element_type=jnp.float32) * dv


def _tc_a(d0, d1, x, xp1, xp2, w1a, w1b):
    return pl.pallas_call(
        _tc_a_body,
        out_shape=[jax.ShapeDtypeStruct((4 * NP, D), jnp.float32),
                   jax.ShapeDtypeStruct((NP, 1), jnp.float32)],
    )(d0, d1, x, xp1, xp2, w1a, w1b)


_RB = 2048
_RG = NP // _RB


def _tc_b_body(pa, pb, h, dinv, bst, ast, wst, o):
    s = pl.program_id(0)
    dv = dinv[...]
    agg = dv * (pa[...] + pb[...] + h[...]) + bst[pl.ds(s, 1), :]
    z = _prelu(agg, ast[pl.ds(s, 1), :])
    wmat = wst[pl.ds(s, 1), :, :][0]
    o[...] = jnp.dot(z, wmat, preferred_element_type=jnp.float32) * dv


def _tc_b(parts, hs, dinv, bst, ast, wst):
    return pl.pallas_call(
        _tc_b_body,
        grid=(4, _RG),
        in_specs=[
            pl.BlockSpec((_RB, D), lambda s, i: (2 * _RG * s + i, 0)),
            pl.BlockSpec((_RB, D), lambda s, i: (2 * _RG * s + _RG + i, 0)),
            pl.BlockSpec((_RB, D), lambda s, i: (_RG * s + i, 0)),
            pl.BlockSpec((_RB, 1), lambda s, i: (i, 0)),
            pl.BlockSpec((4, D), lambda s, i: (0, 0)),
            pl.BlockSpec((4, D), lambda s, i: (0, 0)),
            pl.BlockSpec((4, D, D), lambda s, i: (0, 0, 0)),
        ],
        out_specs=pl.BlockSpec((_RB, D), lambda s, i: (_RG * s + i, 0)),
        out_shape=jax.ShapeDtypeStruct((4 * NP, D), jnp.float32),
    )(parts, parts, hs, dinv, bst, ast, wst)


def _tc_c_body(pa, pb, h, dinv, bst, ast, wp, bp, o, csum, g):
    s = pl.program_id(0)
    i = pl.program_id(1)
    dv = dinv[...]
    agg = dv * (pa[...] + pb[...] + h[...]) + bst[pl.ds(s, 1), :]
    z = _prelu(agg, ast[pl.ds(s, 1), :])
    o[...] = z

    @pl.when(jnp.logical_and(s == 0, i == 0))
    def _():
        csum[...] = jnp.zeros_like(csum)

    rid = lax.broadcasted_iota(jnp.int32, (_RB, 1), 0) + i * _RB
    zm = jnp.where(rid < N, z, 0.0)
    csum[pl.ds(s, 1), :] = (csum[pl.ds(s, 1), :]
                            + jnp.sum(zm, axis=0, keepdims=True))

    @pl.when(jnp.logical_and(s == 1, i == _RG - 1))
    def _():
        m = jax.nn.sigmoid(csum[0:2, :] * (1.0 / N))
        g[...] = (jnp.dot(m, wp[...], preferred_element_type=jnp.float32)
                  + bp[...])


def _tc_c(parts, hs, dinv, bst, ast, wp, bp):
    return pl.pallas_call(
        _tc_c_body,
        grid=(4, _RG),
        in_specs=[
            pl.BlockSpec((_RB, D), lambda s, i: (2 * _RG * s + i, 0)),
            pl.BlockSpec((_RB, D), lambda s, i: (2 * _RG * s + _RG + i, 0)),
            pl.BlockSpec((_RB, D), lambda s, i: (_RG * s + i, 0)),
            pl.BlockSpec((_RB, 1), lambda s, i: (i, 0)),
            pl.BlockSpec((4, D), lambda s, i: (0, 0)),
            pl.BlockSpec((4, D), lambda s, i: (0, 0)),
            pl.BlockSpec((D, D), lambda s, i: (0, 0)),
            pl.BlockSpec((1, D), lambda s, i: (0, 0)),
        ],
        out_specs=[pl.BlockSpec((_RB, D), lambda s, i: (_RG * s + i, 0)),
                   pl.BlockSpec((4, D), lambda s, i: (0, 0)),
                   pl.BlockSpec((2, D), lambda s, i: (0, 0))],
        out_shape=[jax.ShapeDtypeStruct((4 * NP, D), jnp.float32),
                   jax.ShapeDtypeStruct((4, D), jnp.float32),
                   jax.ShapeDtypeStruct((2, D), jnp.float32)],
    )(parts, parts, hs, dinv, bst, ast, wp, bp)


def kernel(x, edge_index, edge_weight, W1a, b1a, W2a, b2a, a1,
           W1b, b1b, W2b, b2b, a2, Wp, bp, perm1, perm2):
    src = edge_index[0].astype(jnp.int32)
    dst = edge_index[1].astype(jnp.int32)
    w = edge_weight.astype(jnp.float32)
    p1 = perm1.astype(jnp.int32)
    p2 = perm2.astype(jnp.int32)

    degflat, xp1, xp2 = _sc_pre(dst, w, x, p1, p2)
    deg2 = degflat.reshape(NC, NP)
    d0 = deg2[0].reshape(NP, 1)
    d1 = deg2[1].reshape(NP, 1)

    pad = jnp.zeros((NP - N, D), jnp.float32)
    xpd = jnp.concatenate([x, pad], axis=0)
    xp1d = jnp.concatenate([xp1, pad], axis=0)
    xp2d = jnp.concatenate([xp2, pad], axis=0)

    hs, dinv = _tc_a(d0, d1, xpd, xp1d, xp2d, W1a, W1b)

    parts1 = _sc_mp(src, dst, w, hs)

    bst1 = jnp.stack([b1a, b1b, b1a, b1b])
    bst2 = jnp.stack([b2a, b2b, b2a, b2b])
    ast = jnp.stack([a1, a2, a1, a2])
    wst2 = jnp.stack([W2a, W2b, W2a, W2b])

    hs2 = _tc_b(parts1, hs, dinv, bst1, ast, wst2)

    parts2 = _sc_mp(src, dst, w, hs2)

    zall, _, g12 = _tc_c(parts2, hs2, dinv, bst2, ast, Wp, bp.reshape(1, D))

    z1 = zall[0:N]
    z2 = zall[NP:NP + N]
    z1n = zall[2 * NP:2 * NP + N]
    z2n = zall[3 * NP:3 * NP + N]
    return (z1, z2, g12[0:1], g12[1:2], z1n, z2n,
            jnp.arange(N, dtype=jnp.int32), x.shape[0])


# speedup vs baseline: 14.6464x; 1.0487x over previous
import functools

import jax
import jax.numpy as jnp
from jax import lax
from jax.experimental import pallas as pl
from jax.experimental.pallas import tpu as pltpu
from jax.experimental.pallas import tpu_sc as plsc

N = 10000
NP = 10240
E = 320000
D = 128
NC = 2
NS = 16
CH = 80
ECHUNKS = E // (NC * NS) // CH
GCHUNKS = N // CH
MCH = 64
MCHUNKS = 312
MTAIL = 32

_MESH = plsc.VectorSubcoreMesh(
    core_axis_name="c", subcore_axis_name="s", num_cores=NC, num_subcores=NS)


def _zero_vmem_rows(buf, nrows):
    z16 = jnp.zeros((16,), jnp.float32)

    def body(i, _):
        for j in range(D // 16):
            buf[i, pl.ds(j * 16, 16)] = z16
        return 0

    lax.fori_loop(0, nrows, body, 0)


DCH = 128
DCHUNKS = 78


@functools.partial(
    pl.kernel,
    out_type=[
        jax.ShapeDtypeStruct((NC * NP,), jnp.float32),
        jax.ShapeDtypeStruct((N, D), jnp.float32),
        jax.ShapeDtypeStruct((N, D), jnp.float32),
    ],
    mesh=_MESH,
    scratch_types=[
        pltpu.VMEM((CH,), jnp.int32),
        pltpu.VMEM((CH,), jnp.float32),
        pltpu.VMEM((CH, D), jnp.float32),
        [pltpu.VMEM((DCH,), jnp.int32) for _ in range(2)],
        [pltpu.VMEM((DCH,), jnp.float32) for _ in range(2)],
        pltpu.VMEM((16,), jnp.int32),
        pltpu.VMEM((16,), jnp.float32),
        pltpu.VMEM_SHARED((NP,), jnp.float32),
        [pltpu.SemaphoreType.DMA for _ in range(2)],
        [pltpu.SemaphoreType.DMA for _ in range(2)],
    ],
)
def _sc_pre(dst_hbm, w_hbm, x_hbm, p1_hbm, p2_hbm,
            deg_hbm, xp1_hbm, xp2_hbm,
            idxb, valb, rows, db, vb, tdb, tvb, dacc, isems, ssems):
    cid = lax.axis_index("c")
    sid = lax.axis_index("s")
    wid = sid * NC + cid
    zslice = NP // NS

    valb[...] = jnp.zeros((CH,), jnp.float32)

    def zbody(i, _):
        pltpu.sync_copy(valb, dacc.at[pl.ds(sid * zslice + i * CH, CH)])
        return 0

    lax.fori_loop(0, zslice // CH, zbody, 0)
    plsc.subcore_barrier()

    base_e = (cid * NS + sid) * (DCHUNKS * DCH + 16)

    def fetch(c, k):
        off = base_e + c * DCH
        pltpu.async_copy(dst_hbm.at[pl.ds(off, DCH)], db[k], isems[k])
        pltpu.async_copy(w_hbm.at[pl.ds(off, DCH)], vb[k], isems[k])

    def wait_fetch(k):
        pltpu.make_async_copy(dst_hbm.at[pl.ds(0, DCH)], db[k],
                              isems[k]).wait()
        pltpu.make_async_copy(w_hbm.at[pl.ds(0, DCH)], vb[k],
                              isems[k]).wait()

    fetch(0, 0)
    fetch(1, 1)

    def ebody(g, _):
        c0 = g * 2
        wait_fetch(0)
        s0 = pltpu.async_copy(vb[0], dacc.at[db[0]], ssems[0], add=True)
        wait_fetch(1)
        s1 = pltpu.async_copy(vb[1], dacc.at[db[1]], ssems[1], add=True)
        s0.wait()
        fetch(jnp.minimum(c0 + 2, DCHUNKS - 1), 0)
        s1.wait()
        fetch(jnp.minimum(c0 + 3, DCHUNKS - 1), 1)
        return 0

    lax.fori_loop(0, DCHUNKS // 2, ebody, 0)
    wait_fetch(0)
    wait_fetch(1)

    toff = base_e + DCHUNKS * DCH
    pltpu.sync_copy(dst_hbm.at[pl.ds(toff, 16)], tdb)
    pltpu.sync_copy(w_hbm.at[pl.ds(toff, 16)], tvb)
    pltpu.sync_copy(tvb, dacc.at[tdb], add=True)
    plsc.subcore_barrier()

    pltpu.sync_copy(dacc.at[pl.ds(sid * zslice, zslice)],
                    deg_hbm.at[pl.ds(cid * NP + sid * zslice, zslice)])

    for i in range((GCHUNKS + NC * NS - 1) // (NC * NS)):
        c = wid + i * NC * NS

        @pl.when(c < GCHUNKS)
        def _():
            off = c * CH
            pltpu.sync_copy(p1_hbm.at[pl.ds(off, CH)], idxb)
            pltpu.sync_copy(x_hbm.at[idxb], rows)
            pltpu.sync_copy(rows, xp1_hbm.at[pl.ds(off, CH)])
            pltpu.sync_copy(p2_hbm.at[pl.ds(off, CH)], idxb)
            pltpu.sync_copy(x_hbm.at[idxb], rows)
            pltpu.sync_copy(rows, xp2_hbm.at[pl.ds(off, CH)])


@functools.partial(
    pl.kernel,
    out_type=jax.ShapeDtypeStruct((4 * NP, D), jnp.float32),
    mesh=_MESH,
    scratch_types=[
        [pltpu.VMEM((MCH,), jnp.int32) for _ in range(6)],
        [pltpu.VMEM((MCH,), jnp.int32) for _ in range(6)],
        [pltpu.VMEM((MCH,), jnp.float32) for _ in range(6)],
        [pltpu.VMEM((MCH, D), jnp.float32) for _ in range(3)],
        pltpu.VMEM((MTAIL,), jnp.int32),
        pltpu.VMEM((MTAIL,), jnp.int32),
        pltpu.VMEM((MTAIL,), jnp.float32),
        pltpu.VMEM((MTAIL, D), jnp.float32),
        pltpu.VMEM_SHARED((NP, D), jnp.float32),
        [pltpu.SemaphoreType.DMA for _ in range(6)],
        [pltpu.SemaphoreType.DMA for _ in range(3)],
        pltpu.SemaphoreType.DMA,
    ],
)
def _sc_mp(src_hbm, dst_hbm, w_hbm, hs_hbm, part_hbm,
           srcb, dstb, wb, rows, tsrc, tdst, tw, trows,
           acc, isems, gsems, ssem):
    cid = lax.axis_index("c")
    sid = lax.axis_index("s")
    zslice = NP // NS
    base_e = sid * (MCHUNKS * MCH + MTAIL)

    def _scale(rows_k, wb_k, nch):
        def sbody(t, _2):
            wvec = wb_k[pl.ds(t * 16, 16)]
            svecs = [jnp.full((16,), wvec[kk], jnp.float32)
                     for kk in range(16)]
            for kk in range(16):
                r = t * 16 + kk
                for j in range(D // 16):
                    sl = pl.ds(j * 16, 16)
                    rows_k[r, sl] = rows_k[r, sl] * svecs[kk]
            return 0

        lax.fori_loop(0, nch // 16, sbody, 0)

    def fetch_idx(c, m):
        off = base_e + c * MCH
        pltpu.async_copy(src_hbm.at[pl.ds(off, MCH)], srcb[m], isems[m])
        pltpu.async_copy(dst_hbm.at[pl.ds(off, MCH)], dstb[m], isems[m])
        pltpu.async_copy(w_hbm.at[pl.ds(off, MCH)], wb[m], isems[m])

    def wait_idx(m):
        pltpu.make_async_copy(src_hbm.at[pl.ds(0, MCH)], srcb[m],
                              isems[m]).wait()
        pltpu.make_async_copy(dst_hbm.at[pl.ds(0, MCH)], dstb[m],
                              isems[m]).wait()
        pltpu.make_async_copy(w_hbm.at[pl.ds(0, MCH)], wb[m],
                              isems[m]).wait()

    def stream_body(sl, _):
        s = cid * 2 + sl
        _zero_vmem_rows(rows[0], MCH)
        zds = [pltpu.async_copy(rows[0],
                                acc.at[pl.ds(sid * zslice + j * MCH, MCH)],
                                gsems[0])
               for j in range(zslice // MCH)]
        for d in zds:
            d.wait()
        plsc.subcore_barrier()

        soff = jnp.full((16,), s * NP, jnp.int32)

        def fire_gather(m, k):
            for j in range(MCH // 16):
                sl = pl.ds(j * 16, 16)
                srcb[m][sl] = srcb[m][sl] + soff
            return pltpu.async_copy(hs_hbm.at[srcb[m]], rows[k], gsems[k])

        for m in range(6):
            fetch_idx(m, m)

        def gbody(g, _g):
            c0 = g * 6
            nxt = c0 + 6
            wait_idx(0)
            gds = [fire_gather(0, 0)]
            wait_idx(1)
            gds.append(fire_gather(1, 1))
            sds = []
            for j in range(6):
                k = j % 3
                gds[j].wait()
                _scale(rows[k], wb[j], MCH)
                sds.append(pltpu.async_copy(rows[k], acc.at[dstb[j]],
                                            ssem, add=True))
                if 1 <= j <= 4:
                    sds[j - 1].wait()
                if j <= 3:
                    wait_idx(j + 2)
                    gds.append(fire_gather(j + 2, (j + 2) % 3))
                if j >= 2:
                    fetch_idx(jnp.minimum(nxt + j - 2, MCHUNKS - 1), j - 2)
            sds[4].wait()
            fetch_idx(jnp.minimum(nxt + 4, MCHUNKS - 1), 4)
            sds[5].wait()
            fetch_idx(jnp.minimum(nxt + 5, MCHUNKS - 1), 5)
            return 0

        lax.fori_loop(0, MCHUNKS // 6, gbody, 0)

        for m in range(6):
            wait_idx(m)

        toff = base_e + MCHUNKS * MCH
        pltpu.sync_copy(src_hbm.at[pl.ds(toff, MTAIL)], tsrc)
        pltpu.sync_copy(dst_hbm.at[pl.ds(toff, MTAIL)], tdst)
        pltpu.sync_copy(w_hbm.at[pl.ds(toff, MTAIL)], tw)
        for j in range(MTAIL // 16):
            tsl = pl.ds(j * 16, 16)
            tsrc[tsl] = tsrc[tsl] + soff
        pltpu.sync_copy(hs_hbm.at[tsrc], trows)
        _scale(trows, tw, MTAIL)
        pltpu.sync_copy(trows, acc.at[tdst], add=True)
        plsc.subcore_barrier()

        dds = []
        for j in range(zslice // MCH):
            roff = sid * zslice + j * MCH
            dds.append(pltpu.async_copy(
                acc.at[pl.ds(roff, MCH)],
                part_hbm.at[pl.ds(s * NP + roff, MCH)],
                gsems[1]))
        for d in dds:
            d.wait()
        plsc.subcore_barrier()
        return 0

    lax.fori_loop(0, 2, stream_body, 0)


def _prelu(z, a):
    return jnp.maximum(z, 0.0) + a * jnp.minimum(z, 0.0)


def _tc_a_body(d0, d1, x, xp1, xp2, w1a, w1b, hs, dinv):
    dv = lax.rsqrt(d0[...] + d1[...] + 1.0)
    dinv[...] = dv
    hs[0:NP, :] = jnp.dot(x[...], w1a[...],
                          preferred_element_type=jnp.float32) * dv
    hs[NP:2 * NP, :] = jnp.dot(x[...], w1b[...],
                               preferred_element_type=jnp.float32) * dv
    hs[2 * NP:3 * NP, :] = jnp.dot(xp1[...], w1a[...],
                                   preferred_element_type=jnp.float32) * dv
    hs[3 * NP:4 * NP, :] = jnp.dot(xp2[...], w1b[...],
                                   preferred_element_type=jnp.float32) * dv


def _tc_a(d0, d1, x, xp1, xp2, w1a, w1b):
    return pl.pallas_call(
        _tc_a_body,
        out_shape=[jax.ShapeDtypeStruct((4 * NP, D), jnp.float32),
                   jax.ShapeDtypeStruct((NP, 1), jnp.float32)],
    )(d0, d1, x, xp1, xp2, w1a, w1b)


_RB = 2048
_RG = NP // _RB


def _tc_b_body(pa, h, dinv, bst, ast, wst, o):
    s = pl.program_id(0)
    dv = dinv[...]
    agg = dv * (pa[...] + h[...]) + bst[pl.ds(s, 1), :]
    z = _prelu(agg, ast[pl.ds(s, 1), :])
    wmat = wst[pl.ds(s, 1), :, :][0]
    o[...] = jnp.dot(z, wmat, preferred_element_type=jnp.float32) * dv


def _tc_b(parts, hs, dinv, bst, ast, wst):
    return pl.pallas_call(
        _tc_b_body,
        grid=(4, _RG),
        in_specs=[
            pl.BlockSpec((_RB, D), lambda s, i: (_RG * s + i, 0)),
            pl.BlockSpec((_RB, D), lambda s, i: (_RG * s + i, 0)),
            pl.BlockSpec((_RB, 1), lambda s, i: (i, 0)),
            pl.BlockSpec((4, D), lambda s, i: (0, 0)),
            pl.BlockSpec((4, D), lambda s, i: (0, 0)),
            pl.BlockSpec((4, D, D), lambda s, i: (0, 0, 0)),
        ],
        out_specs=pl.BlockSpec((_RB, D), lambda s, i: (_RG * s + i, 0)),
        out_shape=jax.ShapeDtypeStruct((4 * NP, D), jnp.float32),
    )(parts, hs, dinv, bst, ast, wst)


def _tc_c_body(pa, h, dinv, bst, ast, wp, bp, o, csum, g):
    s = pl.program_id(0)
    i = pl.program_id(1)
    dv = dinv[...]
    agg = dv * (pa[...] + h[...]) + bst[pl.ds(s, 1), :]
    z = _prelu(agg, ast[pl.ds(s, 1), :])
    o[...] = z

    @pl.when(jnp.logical_and(s == 0, i == 0))
    def _():
        csum[...] = jnp.zeros_like(csum)

    rid = lax.broadcasted_iota(jnp.int32, (_RB, 1), 0) + i * _RB
    zm = jnp.where(rid < N, z, 0.0)
    csum[pl.ds(s, 1), :] = (csum[pl.ds(s, 1), :]
                            + jnp.sum(zm, axis=0, keepdims=True))

    @pl.when(jnp.logical_and(s == 1, i == _RG - 1))
    def _():
        m = jax.nn.sigmoid(csum[0:2, :] * (1.0 / N))
        g[...] = (jnp.dot(m, wp[...], preferred_element_type=jnp.float32)
                  + bp[...])


def _tc_c(parts, hs, dinv, bst, ast, wp, bp):
    return pl.pallas_call(
        _tc_c_body,
        grid=(4, _RG),
        in_specs=[
            pl.BlockSpec((_RB, D), lambda s, i: (_RG * s + i, 0)),
            pl.BlockSpec((_RB, D), lambda s, i: (_RG * s + i, 0)),
            pl.BlockSpec((_RB, 1), lambda s, i: (i, 0)),
            pl.BlockSpec((4, D), lambda s, i: (0, 0)),
            pl.BlockSpec((4, D), lambda s, i: (0, 0)),
            pl.BlockSpec((D, D), lambda s, i: (0, 0)),
            pl.BlockSpec((1, D), lambda s, i: (0, 0)),
        ],
        out_specs=[pl.BlockSpec((_RB, D), lambda s, i: (_RG * s + i, 0)),
                   pl.BlockSpec((4, D), lambda s, i: (0, 0)),
                   pl.BlockSpec((2, D), lambda s, i: (0, 0))],
        out_shape=[jax.ShapeDtypeStruct((4 * NP, D), jnp.float32),
                   jax.ShapeDtypeStruct((4, D), jnp.float32),
                   jax.ShapeDtypeStruct((2, D), jnp.float32)],
    )(parts, hs, dinv, bst, ast, wp, bp)


def kernel(x, edge_index, edge_weight, W1a, b1a, W2a, b2a, a1,
           W1b, b1b, W2b, b2b, a2, Wp, bp, perm1, perm2):
    src = edge_index[0].astype(jnp.int32)
    dst = edge_index[1].astype(jnp.int32)
    w = edge_weight.astype(jnp.float32)
    p1 = perm1.astype(jnp.int32)
    p2 = perm2.astype(jnp.int32)

    degflat, xp1, xp2 = _sc_pre(dst, w, x, p1, p2)
    deg2 = degflat.reshape(NC, NP)
    d0 = deg2[0].reshape(NP, 1)
    d1 = deg2[1].reshape(NP, 1)

    pad = jnp.zeros((NP - N, D), jnp.float32)
    xpd = jnp.concatenate([x, pad], axis=0)
    xp1d = jnp.concatenate([xp1, pad], axis=0)
    xp2d = jnp.concatenate([xp2, pad], axis=0)

    hs, dinv = _tc_a(d0, d1, xpd, xp1d, xp2d, W1a, W1b)

    parts1 = _sc_mp(src, dst, w, hs)

    bst1 = jnp.stack([b1a, b1b, b1a, b1b])
    bst2 = jnp.stack([b2a, b2b, b2a, b2b])
    ast = jnp.stack([a1, a2, a1, a2])
    wst2 = jnp.stack([W2a, W2b, W2a, W2b])

    hs2 = _tc_b(parts1, hs, dinv, bst1, ast, wst2)

    parts2 = _sc_mp(src, dst, w, hs2)

    zall, _, g12 = _tc_c(parts2, hs2, dinv, bst2, ast, Wp, bp.reshape(1, D))

    z1 = zall[0:N]
    z2 = zall[NP:NP + N]
    z1n = zall[2 * NP:2 * NP + N]
    z2n = zall[3 * NP:3 * NP + N]
    return (z1, z2, g12[0:1], g12[1:2], z1n, z2n,
            jnp.arange(N, dtype=jnp.int32), x.shape[0])
